# Initial kernel scaffold; baseline (speedup 1.0000x reference)
import jax
import jax.numpy as jnp
from jax.experimental import pallas as pl


def kernel(x, edge_index, batch, W0, b0, W1, b1, W2, b2, lin_W, lin_b):
    raise NotImplementedError("write your pallas kernel here")


# speedup vs baseline: 11.8191x; 11.8191x over previous
import jax
import jax.numpy as jnp
from jax import lax
from jax.experimental import pallas as pl
from jax.experimental.pallas import tpu as pltpu
from jax.experimental.pallas import tpu_sc as plsc

N = 10000
E = 320000
D = 128
G = 64
P = 16

NC = 2
NS = 16
NW = NC * NS
EPW = E // NW
CHUNK = 80
NCH = EPW // CHUNK
SPLIT = 624
LAST = N - SPLIT * (NS - 1)

R = 1000
NBLK = N // R


def _zero_acc(sid, zbuf, acc):
    @pl.loop(0, pl.cdiv(N // CHUNK, NS))
    def _(k):
        c = k * NS + sid

        @pl.when(c < N // CHUNK)
        def _():
            pltpu.sync_copy(zbuf, acc.at[pl.ds(c * CHUNK, CHUNK)])


def _drain_acc(cid, sid, acc, parts_hbm):
    @pl.when(sid < NS - 1)
    def _():
        pltpu.sync_copy(acc.at[pl.ds(sid * SPLIT, SPLIT)],
                        parts_hbm.at[cid, pl.ds(sid * SPLIT, SPLIT)])

    @pl.when(sid == NS - 1)
    def _():
        pltpu.sync_copy(acc.at[pl.ds((NS - 1) * SPLIT, LAST)],
                        parts_hbm.at[cid, pl.ds((NS - 1) * SPLIT, LAST)])


def _deg_body(dst_hbm, parts_hbm, acc, didx, ones_v):
    cid = lax.axis_index("c")
    sid = lax.axis_index("s")
    w = sid * NC + cid

    @pl.loop(0, CHUNK)
    def _zero(i):
        for j in range(D // 16):
            ones_v[i, pl.ds(j * 16, 16)] = jnp.zeros((16,), jnp.float32)

    _zero_acc(sid, ones_v, acc)

    @pl.loop(0, CHUNK)
    def _fill(i):
        for j in range(D // 16):
            ones_v[i, pl.ds(j * 16, 16)] = jnp.full((16,), 1.0, jnp.float32)

    plsc.subcore_barrier()

    @pl.loop(0, NCH)
    def _step(it):
        e0 = w * EPW + it * CHUNK
        pltpu.sync_copy(dst_hbm.at[pl.ds(e0, CHUNK)], didx)
        pltpu.sync_copy(ones_v, acc.at[didx], add=True)

    plsc.subcore_barrier()
    _drain_acc(cid, sid, acc, parts_hbm)


_deg = pl.kernel(
    _deg_body,
    out_type=jax.ShapeDtypeStruct((NC, N, D), jnp.float32),
    mesh=plsc.VectorSubcoreMesh(core_axis_name="c", subcore_axis_name="s"),
    compiler_params=pltpu.CompilerParams(use_tc_tiling_on_sc=False),
    scratch_types=[
        pltpu.VMEM_SHARED((N, D), jnp.float32),
        pltpu.VMEM((CHUNK,), jnp.int32),
        pltpu.VMEM((CHUNK, D), jnp.float32),
    ],
)


def _agg_body(u_hbm, src_hbm, dst_hbm, parts_hbm, acc, sidx, didx, rows,
              gsem):
    cid = lax.axis_index("c")
    sid = lax.axis_index("s")
    w = sid * NC + cid

    @pl.loop(0, CHUNK)
    def _zero(i):
        for j in range(D // 16):
            rows[i, pl.ds(j * 16, 16)] = jnp.zeros((16,), jnp.float32)

    _zero_acc(sid, rows, acc)
    plsc.subcore_barrier()

    @pl.loop(0, NCH)
    def _step(it):
        e0 = w * EPW + it * CHUNK
        pltpu.sync_copy(src_hbm.at[pl.ds(e0, CHUNK)], sidx)
        cp = pltpu.async_copy(u_hbm.at[sidx], rows, gsem)
        pltpu.sync_copy(dst_hbm.at[pl.ds(e0, CHUNK)], didx)
        cp.wait()
        pltpu.sync_copy(rows, acc.at[didx], add=True)

    plsc.subcore_barrier()
    _drain_acc(cid, sid, acc, parts_hbm)


_agg = pl.kernel(
    _agg_body,
    out_type=jax.ShapeDtypeStruct((NC, N, D), jnp.float32),
    mesh=plsc.VectorSubcoreMesh(core_axis_name="c", subcore_axis_name="s"),
    compiler_params=pltpu.CompilerParams(use_tc_tiling_on_sc=False),
    scratch_types=[
        pltpu.VMEM_SHARED((N, D), jnp.float32),
        pltpu.VMEM((CHUNK,), jnp.int32),
        pltpu.VMEM((CHUNK,), jnp.int32),
        pltpu.VMEM((CHUNK, D), jnp.float32),
        pltpu.SemaphoreType.DMA,
    ],
)


def _mm0_body(x_ref, degp_ref, w_ref, u_ref, dinv_ref):
    dp = degp_ref[...]
    deg = dp[0, :, 0:1] + dp[1, :, 0:1] + 1.0
    dinv = lax.rsqrt(deg)
    h = jnp.dot(x_ref[...], w_ref[...], preferred_element_type=jnp.float32)
    u_ref[...] = dinv * h
    dinv_ref[...] = jnp.broadcast_to(dinv, (R, D))


_mm0 = pl.pallas_call(
    _mm0_body,
    grid=(NBLK,),
    in_specs=[
        pl.BlockSpec((R, D), lambda i: (i, 0)),
        pl.BlockSpec((NC, R, D), lambda i: (0, i, 0)),
        pl.BlockSpec((D, D), lambda i: (0, 0)),
    ],
    out_specs=[
        pl.BlockSpec((R, D), lambda i: (i, 0)),
        pl.BlockSpec((R, D), lambda i: (i, 0)),
    ],
    out_shape=[
        jax.ShapeDtypeStruct((N, D), jnp.float32),
        jax.ShapeDtypeStruct((N, D), jnp.float32),
    ],
)


def _mm_body(p_ref, u_ref, dinv_ref, b_ref, w_ref, out_ref):
    p = p_ref[...]
    dinv = dinv_ref[...]
    a = jnp.maximum(dinv * (p[0] + p[1] + u_ref[...]) + b_ref[...], 0.0)
    out_ref[...] = dinv * jnp.dot(a, w_ref[...],
                                  preferred_element_type=jnp.float32)


_mm = pl.pallas_call(
    _mm_body,
    grid=(NBLK,),
    in_specs=[
        pl.BlockSpec((NC, R, D), lambda i: (0, i, 0)),
        pl.BlockSpec((R, D), lambda i: (i, 0)),
        pl.BlockSpec((R, D), lambda i: (i, 0)),
        pl.BlockSpec((1, D), lambda i: (0, 0)),
        pl.BlockSpec((D, D), lambda i: (0, 0)),
    ],
    out_specs=pl.BlockSpec((R, D), lambda i: (i, 0)),
    out_shape=jax.ShapeDtypeStruct((N, D), jnp.float32),
)


def _pool_body(p_ref, u_ref, dinv_ref, b_ref, batch_ref, lw_ref, lb_ref,
               out_ref, pooled):
    i = pl.program_id(0)
    p = p_ref[...]
    dinv = dinv_ref[...]
    a = jnp.maximum(dinv * (p[0] + p[1] + u_ref[...]) + b_ref[...], 0.0)
    bid = batch_ref[...]

    @pl.when(i == 0)
    def _():
        pooled[...] = jnp.full((G, D), -jnp.inf, jnp.float32)

    rows = []
    for j in range(G):
        mj = jnp.max(jnp.where(bid == j, a, -jnp.inf), axis=0, keepdims=True)
        rows.append(mj)
    pooled[...] = jnp.maximum(pooled[...], jnp.concatenate(rows, axis=0))

    @pl.when(i == NBLK - 1)
    def _():
        out_ref[...] = (jnp.dot(pooled[...], lw_ref[...],
                                preferred_element_type=jnp.float32)
                        + lb_ref[...])


_pool = pl.pallas_call(
    _pool_body,
    grid=(NBLK,),
    in_specs=[
        pl.BlockSpec((NC, R, D), lambda i: (0, i, 0)),
        pl.BlockSpec((R, D), lambda i: (i, 0)),
        pl.BlockSpec((R, D), lambda i: (i, 0)),
        pl.BlockSpec((1, D), lambda i: (0, 0)),
        pl.BlockSpec((R, 1), lambda i: (i, 0)),
        pl.BlockSpec((D, P), lambda i: (0, 0)),
        pl.BlockSpec((1, P), lambda i: (0, 0)),
    ],
    out_specs=pl.BlockSpec((G, P), lambda i: (0, 0)),
    out_shape=jax.ShapeDtypeStruct((G, P), jnp.float32),
    scratch_shapes=[pltpu.VMEM((G, D), jnp.float32)],
)


def kernel(x, edge_index, batch, W0, b0, W1, b1, W2, b2, lin_W, lin_b):
    src = edge_index[0]
    dst = edge_index[1]
    degp = _deg(dst)
    u0, dinv = _mm0(x, degp, W0)
    g0 = _agg(u0, src, dst)
    u1 = _mm(g0, u0, dinv, b0.reshape(1, D), W1)
    g1 = _agg(u1, src, dst)
    u2 = _mm(g1, u1, dinv, b1.reshape(1, D), W2)
    g2 = _agg(u2, src, dst)
    out = _pool(g2, u2, dinv, b2.reshape(1, D), batch.reshape(N, 1),
                lin_W, lin_b.reshape(1, P))
    return out


# speedup vs baseline: 21.3439x; 1.8059x over previous
import jax
import jax.numpy as jnp
from jax import lax
from jax.experimental import pallas as pl
from jax.experimental.pallas import tpu as pltpu
from jax.experimental.pallas import tpu_sc as plsc

N = 10000
E = 320000
D = 128
G = 64
P = 16

NC = 2
NS = 16
NW = NC * NS
EPW = E // NW
CHUNK = 80
NCH = EPW // CHUNK
SPLIT = 624
LAST = N - SPLIT * (NS - 1)

R = 1000
NBLK = N // R


def _zero_acc(sid, zbuf, acc):
    @pl.loop(0, pl.cdiv(N // CHUNK, NS))
    def _(k):
        c = k * NS + sid

        @pl.when(c < N // CHUNK)
        def _():
            pltpu.sync_copy(zbuf, acc.at[pl.ds(c * CHUNK, CHUNK)])


def _drain_acc(cid, sid, acc, parts_hbm):
    @pl.when(sid < NS - 1)
    def _():
        pltpu.sync_copy(acc.at[pl.ds(sid * SPLIT, SPLIT)],
                        parts_hbm.at[cid, pl.ds(sid * SPLIT, SPLIT)])

    @pl.when(sid == NS - 1)
    def _():
        pltpu.sync_copy(acc.at[pl.ds((NS - 1) * SPLIT, LAST)],
                        parts_hbm.at[cid, pl.ds((NS - 1) * SPLIT, LAST)])


def _deg_body(dst_hbm, parts_hbm, acc, didx_all, ones_v):
    cid = lax.axis_index("c")
    sid = lax.axis_index("s")
    w = sid * NC + cid

    @pl.loop(0, CHUNK)
    def _zero(i):
        for j in range(D // 16):
            ones_v[i, pl.ds(j * 16, 16)] = jnp.zeros((16,), jnp.float32)

    _zero_acc(sid, ones_v, acc)
    pltpu.sync_copy(dst_hbm.at[w], didx_all)

    @pl.loop(0, CHUNK)
    def _fill(i):
        for j in range(D // 16):
            ones_v[i, pl.ds(j * 16, 16)] = jnp.full((16,), 1.0, jnp.float32)

    plsc.subcore_barrier()

    @pl.loop(0, NCH)
    def _step(it):
        pltpu.sync_copy(ones_v, acc.at[didx_all.at[it]], add=True)

    plsc.subcore_barrier()
    _drain_acc(cid, sid, acc, parts_hbm)


_deg = pl.kernel(
    _deg_body,
    out_type=jax.ShapeDtypeStruct((NC, N, D), jnp.float32),
    mesh=plsc.VectorSubcoreMesh(core_axis_name="c", subcore_axis_name="s"),
    compiler_params=pltpu.CompilerParams(use_tc_tiling_on_sc=False),
    scratch_types=[
        pltpu.VMEM_SHARED((N, D), jnp.float32),
        pltpu.VMEM((NCH, CHUNK), jnp.int32),
        pltpu.VMEM((CHUNK, D), jnp.float32),
    ],
)


def _agg_body(u_hbm, src_hbm, dst_hbm, parts_hbm, acc, sidx_all, didx_all,
              rows0, rows1, sem0, sem1):
    cid = lax.axis_index("c")
    sid = lax.axis_index("s")
    w = sid * NC + cid

    @pl.loop(0, CHUNK)
    def _zero(i):
        for j in range(D // 16):
            rows0[i, pl.ds(j * 16, 16)] = jnp.zeros((16,), jnp.float32)

    _zero_acc(sid, rows0, acc)
    pltpu.sync_copy(src_hbm.at[w], sidx_all)
    pltpu.sync_copy(dst_hbm.at[w], didx_all)
    plsc.subcore_barrier()

    rows = (rows0, rows1)
    sems = (sem0, sem1)

    pltpu.async_copy(u_hbm.at[sidx_all.at[0]], rows0, sem0)

    @pl.loop(0, (NCH + 1) // 2)
    def _step(k):
        c0 = 2 * k
        for b in range(2):
            c = c0 + b
            nxt = c + 1

            @pl.when(nxt < NCH)
            def _():
                pltpu.async_copy(u_hbm.at[sidx_all.at[nxt]],
                                 rows[(b + 1) % 2], sems[(b + 1) % 2])

            @pl.when(c < NCH)
            def _():
                pltpu.make_async_copy(u_hbm.at[sidx_all.at[c]],
                                      rows[b], sems[b]).wait()
                pltpu.sync_copy(rows[b], acc.at[didx_all.at[c]], add=True)

    plsc.subcore_barrier()
    _drain_acc(cid, sid, acc, parts_hbm)


_agg = pl.kernel(
    _agg_body,
    out_type=jax.ShapeDtypeStruct((NC, N, D), jnp.float32),
    mesh=plsc.VectorSubcoreMesh(core_axis_name="c", subcore_axis_name="s"),
    compiler_params=pltpu.CompilerParams(use_tc_tiling_on_sc=False),
    scratch_types=[
        pltpu.VMEM_SHARED((N, D), jnp.float32),
        pltpu.VMEM((NCH, CHUNK), jnp.int32),
        pltpu.VMEM((NCH, CHUNK), jnp.int32),
        pltpu.VMEM((CHUNK, D), jnp.float32),
        pltpu.VMEM((CHUNK, D), jnp.float32),
        pltpu.SemaphoreType.DMA,
        pltpu.SemaphoreType.DMA,
    ],
)


def _mm0_body(x_ref, degp_ref, w_ref, u_ref, dinv_ref):
    dp = degp_ref[...]
    deg = dp[0, :, 0:1] + dp[1, :, 0:1] + 1.0
    dinv = lax.rsqrt(deg)
    h = jnp.dot(x_ref[...], w_ref[...], preferred_element_type=jnp.float32)
    u_ref[...] = dinv * h
    dinv_ref[...] = jnp.broadcast_to(dinv, (R, D))


_mm0 = pl.pallas_call(
    _mm0_body,
    grid=(NBLK,),
    in_specs=[
        pl.BlockSpec((R, D), lambda i: (i, 0)),
        pl.BlockSpec((NC, R, D), lambda i: (0, i, 0)),
        pl.BlockSpec((D, D), lambda i: (0, 0)),
    ],
    out_specs=[
        pl.BlockSpec((R, D), lambda i: (i, 0)),
        pl.BlockSpec((R, D), lambda i: (i, 0)),
    ],
    out_shape=[
        jax.ShapeDtypeStruct((N, D), jnp.float32),
        jax.ShapeDtypeStruct((N, D), jnp.float32),
    ],
)


def _mm_body(p_ref, u_ref, dinv_ref, b_ref, w_ref, out_ref):
    p = p_ref[...]
    dinv = dinv_ref[...]
    a = jnp.maximum(dinv * (p[0] + p[1] + u_ref[...]) + b_ref[...], 0.0)
    out_ref[...] = dinv * jnp.dot(a, w_ref[...],
                                  preferred_element_type=jnp.float32)


_mm = pl.pallas_call(
    _mm_body,
    grid=(NBLK,),
    in_specs=[
        pl.BlockSpec((NC, R, D), lambda i: (0, i, 0)),
        pl.BlockSpec((R, D), lambda i: (i, 0)),
        pl.BlockSpec((R, D), lambda i: (i, 0)),
        pl.BlockSpec((1, D), lambda i: (0, 0)),
        pl.BlockSpec((D, D), lambda i: (0, 0)),
    ],
    out_specs=pl.BlockSpec((R, D), lambda i: (i, 0)),
    out_shape=jax.ShapeDtypeStruct((N, D), jnp.float32),
)


def _pool_body(p_ref, u_ref, dinv_ref, b_ref, batch_ref, lw_ref, lb_ref,
               out_ref, pooled):
    i = pl.program_id(0)
    p = p_ref[...]
    dinv = dinv_ref[...]
    a = jnp.maximum(dinv * (p[0] + p[1] + u_ref[...]) + b_ref[...], 0.0)
    bid = batch_ref[...]

    @pl.when(i == 0)
    def _():
        pooled[...] = jnp.full((G, D), -jnp.inf, jnp.float32)

    rows = []
    for j in range(G):
        mj = jnp.max(jnp.where(bid == j, a, -jnp.inf), axis=0, keepdims=True)
        rows.append(mj)
    pooled[...] = jnp.maximum(pooled[...], jnp.concatenate(rows, axis=0))

    @pl.when(i == NBLK - 1)
    def _():
        out_ref[...] = (jnp.dot(pooled[...], lw_ref[...],
                                preferred_element_type=jnp.float32)
                        + lb_ref[...])


_pool = pl.pallas_call(
    _pool_body,
    grid=(NBLK,),
    in_specs=[
        pl.BlockSpec((NC, R, D), lambda i: (0, i, 0)),
        pl.BlockSpec((R, D), lambda i: (i, 0)),
        pl.BlockSpec((R, D), lambda i: (i, 0)),
        pl.BlockSpec((1, D), lambda i: (0, 0)),
        pl.BlockSpec((R, 1), lambda i: (i, 0)),
        pl.BlockSpec((D, P), lambda i: (0, 0)),
        pl.BlockSpec((1, P), lambda i: (0, 0)),
    ],
    out_specs=pl.BlockSpec((G, P), lambda i: (0, 0)),
    out_shape=jax.ShapeDtypeStruct((G, P), jnp.float32),
    scratch_shapes=[pltpu.VMEM((G, D), jnp.float32)],
)


def kernel(x, edge_index, batch, W0, b0, W1, b1, W2, b2, lin_W, lin_b):
    src = edge_index[0].reshape(NW, NCH, CHUNK)
    dst = edge_index[1].reshape(NW, NCH, CHUNK)
    degp = _deg(dst)
    u0, dinv = _mm0(x, degp, W0)
    g0 = _agg(u0, src, dst)
    u1 = _mm(g0, u0, dinv, b0.reshape(1, D), W1)
    g1 = _agg(u1, src, dst)
    u2 = _mm(g1, u1, dinv, b1.reshape(1, D), W2)
    g2 = _agg(u2, src, dst)
    out = _pool(g2, u2, dinv, b2.reshape(1, D), batch.reshape(N, 1),
                lin_W, lin_b.reshape(1, P))
    return out


# speedup vs baseline: 22.9684x; 1.0761x over previous
import jax
import jax.numpy as jnp
from jax import lax
from jax.experimental import pallas as pl
from jax.experimental.pallas import tpu as pltpu
from jax.experimental.pallas import tpu_sc as plsc

N = 10000
E = 320000
D = 128
G = 64
P = 16

NC = 2
NS = 16
NW = NC * NS
EPW = E // NW
CHUNK = 80
NCH = EPW // CHUNK
SPLIT = 624
LAST = N - SPLIT * (NS - 1)

R = 1000
NBLK = N // R


def _zero_acc(sid, zbuf, acc):
    @pl.loop(0, pl.cdiv(N // CHUNK, NS))
    def _(k):
        c = k * NS + sid

        @pl.when(c < N // CHUNK)
        def _():
            pltpu.sync_copy(zbuf, acc.at[pl.ds(c * CHUNK, CHUNK)])


def _drain_acc(cid, sid, acc, parts_hbm):
    @pl.when(sid < NS - 1)
    def _():
        pltpu.sync_copy(acc.at[pl.ds(sid * SPLIT, SPLIT)],
                        parts_hbm.at[cid, pl.ds(sid * SPLIT, SPLIT)])

    @pl.when(sid == NS - 1)
    def _():
        pltpu.sync_copy(acc.at[pl.ds((NS - 1) * SPLIT, LAST)],
                        parts_hbm.at[cid, pl.ds((NS - 1) * SPLIT, LAST)])


DW = 16


def _deg_body(dst_hbm, parts_hbm, acc, didx_all, ones_v):
    cid = lax.axis_index("c")
    sid = lax.axis_index("s")
    w = sid * NC + cid

    @pl.loop(0, CHUNK)
    def _zero(i):
        ones_v[i] = jnp.zeros((DW,), jnp.float32)

    _zero_acc(sid, ones_v, acc)
    pltpu.sync_copy(dst_hbm.at[w], didx_all)

    @pl.loop(0, CHUNK)
    def _fill(i):
        ones_v[i] = jnp.full((DW,), 1.0, jnp.float32)

    plsc.subcore_barrier()

    @pl.loop(0, NCH)
    def _step(it):
        pltpu.sync_copy(ones_v, acc.at[didx_all.at[it]], add=True)

    plsc.subcore_barrier()
    _drain_acc(cid, sid, acc, parts_hbm)


_deg = pl.kernel(
    _deg_body,
    out_type=jax.ShapeDtypeStruct((NC, N, DW), jnp.float32),
    mesh=plsc.VectorSubcoreMesh(core_axis_name="c", subcore_axis_name="s"),
    compiler_params=pltpu.CompilerParams(use_tc_tiling_on_sc=False),
    scratch_types=[
        pltpu.VMEM_SHARED((N, DW), jnp.float32),
        pltpu.VMEM((NCH, CHUNK), jnp.int32),
        pltpu.VMEM((CHUNK, DW), jnp.float32),
    ],
)


def _agg_body(u_hbm, src_hbm, dst_hbm, parts_hbm, acc, sidx_all, didx_all,
              rows0, rows1, sem0, sem1):
    cid = lax.axis_index("c")
    sid = lax.axis_index("s")
    w = sid * NC + cid

    @pl.loop(0, CHUNK)
    def _zero(i):
        for j in range(D // 16):
            rows0[i, pl.ds(j * 16, 16)] = jnp.zeros((16,), jnp.float32)

    _zero_acc(sid, rows0, acc)
    pltpu.sync_copy(src_hbm.at[w], sidx_all)
    pltpu.sync_copy(dst_hbm.at[w], didx_all)
    plsc.subcore_barrier()

    rows = (rows0, rows1)
    sems = (sem0, sem1)

    pltpu.async_copy(u_hbm.at[sidx_all.at[0]], rows0, sem0)

    @pl.loop(0, (NCH + 1) // 2)
    def _step(k):
        c0 = 2 * k
        for b in range(2):
            c = c0 + b
            nxt = c + 1

            @pl.when(nxt < NCH)
            def _():
                pltpu.async_copy(u_hbm.at[sidx_all.at[nxt]],
                                 rows[(b + 1) % 2], sems[(b + 1) % 2])

            @pl.when(c < NCH)
            def _():
                pltpu.make_async_copy(u_hbm.at[sidx_all.at[c]],
                                      rows[b], sems[b]).wait()
                pltpu.sync_copy(rows[b], acc.at[didx_all.at[c]], add=True)

    plsc.subcore_barrier()
    _drain_acc(cid, sid, acc, parts_hbm)


_agg = pl.kernel(
    _agg_body,
    out_type=jax.ShapeDtypeStruct((NC, N, D), jnp.float32),
    mesh=plsc.VectorSubcoreMesh(core_axis_name="c", subcore_axis_name="s"),
    compiler_params=pltpu.CompilerParams(use_tc_tiling_on_sc=False),
    scratch_types=[
        pltpu.VMEM_SHARED((N, D), jnp.float32),
        pltpu.VMEM((NCH, CHUNK), jnp.int32),
        pltpu.VMEM((NCH, CHUNK), jnp.int32),
        pltpu.VMEM((CHUNK, D), jnp.float32),
        pltpu.VMEM((CHUNK, D), jnp.float32),
        pltpu.SemaphoreType.DMA,
        pltpu.SemaphoreType.DMA,
    ],
)


def _mm0_body(x_ref, degp_ref, w_ref, u_ref, dinv_ref):
    dp = degp_ref[...]
    deg = dp[0, :, 0:1] + dp[1, :, 0:1] + 1.0
    dinv = lax.rsqrt(deg)
    h = jnp.dot(x_ref[...], w_ref[...], preferred_element_type=jnp.float32)
    u_ref[...] = dinv * h
    dinv_ref[...] = dinv


_mm0 = pl.pallas_call(
    _mm0_body,
    grid=(NBLK,),
    in_specs=[
        pl.BlockSpec((R, D), lambda i: (i, 0)),
        pl.BlockSpec((NC, R, DW), lambda i: (0, i, 0)),
        pl.BlockSpec((D, D), lambda i: (0, 0)),
    ],
    out_specs=[
        pl.BlockSpec((R, D), lambda i: (i, 0)),
        pl.BlockSpec((R, 1), lambda i: (i, 0)),
    ],
    out_shape=[
        jax.ShapeDtypeStruct((N, D), jnp.float32),
        jax.ShapeDtypeStruct((N, 1), jnp.float32),
    ],
)


def _mm_body(p_ref, u_ref, dinv_ref, b_ref, w_ref, out_ref):
    p = p_ref[...]
    dinv = dinv_ref[...]
    a = jnp.maximum(dinv * (p[0] + p[1] + u_ref[...]) + b_ref[...], 0.0)
    out_ref[...] = dinv * jnp.dot(a, w_ref[...],
                                  preferred_element_type=jnp.float32)


_mm = pl.pallas_call(
    _mm_body,
    grid=(NBLK,),
    in_specs=[
        pl.BlockSpec((NC, R, D), lambda i: (0, i, 0)),
        pl.BlockSpec((R, D), lambda i: (i, 0)),
        pl.BlockSpec((R, 1), lambda i: (i, 0)),
        pl.BlockSpec((1, D), lambda i: (0, 0)),
        pl.BlockSpec((D, D), lambda i: (0, 0)),
    ],
    out_specs=pl.BlockSpec((R, D), lambda i: (i, 0)),
    out_shape=jax.ShapeDtypeStruct((N, D), jnp.float32),
)


def _pool_body(p_ref, u_ref, dinv_ref, b_ref, batch_ref, lw_ref, lb_ref,
               out_ref, pooled):
    i = pl.program_id(0)
    p = p_ref[...]
    dinv = dinv_ref[...]
    a = jnp.maximum(dinv * (p[0] + p[1] + u_ref[...]) + b_ref[...], 0.0)
    bid = batch_ref[...]

    @pl.when(i == 0)
    def _():
        pooled[...] = jnp.full((G, D), -jnp.inf, jnp.float32)

    rows = []
    for j in range(G):
        mj = jnp.max(jnp.where(bid == j, a, -jnp.inf), axis=0, keepdims=True)
        rows.append(mj)
    pooled[...] = jnp.maximum(pooled[...], jnp.concatenate(rows, axis=0))

    @pl.when(i == NBLK - 1)
    def _():
        out_ref[...] = (jnp.dot(pooled[...], lw_ref[...],
                                preferred_element_type=jnp.float32)
                        + lb_ref[...])


_pool = pl.pallas_call(
    _pool_body,
    grid=(NBLK,),
    in_specs=[
        pl.BlockSpec((NC, R, D), lambda i: (0, i, 0)),
        pl.BlockSpec((R, D), lambda i: (i, 0)),
        pl.BlockSpec((R, 1), lambda i: (i, 0)),
        pl.BlockSpec((1, D), lambda i: (0, 0)),
        pl.BlockSpec((R, 1), lambda i: (i, 0)),
        pl.BlockSpec((D, P), lambda i: (0, 0)),
        pl.BlockSpec((1, P), lambda i: (0, 0)),
    ],
    out_specs=pl.BlockSpec((G, P), lambda i: (0, 0)),
    out_shape=jax.ShapeDtypeStruct((G, P), jnp.float32),
    scratch_shapes=[pltpu.VMEM((G, D), jnp.float32)],
)


def kernel(x, edge_index, batch, W0, b0, W1, b1, W2, b2, lin_W, lin_b):
    src = edge_index[0].reshape(NW, NCH, CHUNK)
    dst = edge_index[1].reshape(NW, NCH, CHUNK)
    degp = _deg(dst)
    u0, dinv = _mm0(x, degp, W0)
    g0 = _agg(u0, src, dst)
    u1 = _mm(g0, u0, dinv, b0.reshape(1, D), W1)
    g1 = _agg(u1, src, dst)
    u2 = _mm(g1, u1, dinv, b1.reshape(1, D), W2)
    g2 = _agg(u2, src, dst)
    out = _pool(g2, u2, dinv, b2.reshape(1, D), batch.reshape(N, 1),
                lin_W, lin_b.reshape(1, P))
    return out


# speedup vs baseline: 26.0101x; 1.1324x over previous
import jax
import jax.numpy as jnp
from jax import lax
from jax.experimental import pallas as pl
from jax.experimental.pallas import tpu as pltpu
from jax.experimental.pallas import tpu_sc as plsc

N = 10000
E = 320000
D = 128
G = 64
P = 16

NC = 2
NS = 16
NW = NC * NS
EPW = E // NW
CHUNK = 80
NCH = EPW // CHUNK
SPLIT = 624
LAST = N - SPLIT * (NS - 1)

R = 1000
NBLK = N // R


def _zero_acc(sid, zbuf, acc):
    @pl.loop(0, pl.cdiv(N // CHUNK, NS))
    def _(k):
        c = k * NS + sid

        @pl.when(c < N // CHUNK)
        def _():
            pltpu.sync_copy(zbuf, acc.at[pl.ds(c * CHUNK, CHUNK)])


def _drain_acc(cid, sid, acc, parts_hbm):
    @pl.when(sid < NS - 1)
    def _():
        pltpu.sync_copy(acc.at[pl.ds(sid * SPLIT, SPLIT)],
                        parts_hbm.at[cid, pl.ds(sid * SPLIT, SPLIT)])

    @pl.when(sid == NS - 1)
    def _():
        pltpu.sync_copy(acc.at[pl.ds((NS - 1) * SPLIT, LAST)],
                        parts_hbm.at[cid, pl.ds((NS - 1) * SPLIT, LAST)])


DW = 16


def _deg_body(dst_hbm, parts_hbm, acc, didx_all, ones_v):
    cid = lax.axis_index("c")
    sid = lax.axis_index("s")
    w = sid * NC + cid

    @pl.loop(0, CHUNK)
    def _zero(i):
        ones_v[i] = jnp.zeros((DW,), jnp.float32)

    _zero_acc(sid, ones_v, acc)
    pltpu.sync_copy(dst_hbm.at[w], didx_all)

    @pl.loop(0, CHUNK)
    def _fill(i):
        ones_v[i] = jnp.full((DW,), 1.0, jnp.float32)

    plsc.subcore_barrier()

    @pl.loop(0, NCH)
    def _step(it):
        pltpu.sync_copy(ones_v, acc.at[didx_all.at[it]], add=True)

    plsc.subcore_barrier()
    _drain_acc(cid, sid, acc, parts_hbm)


_deg = pl.kernel(
    _deg_body,
    out_type=jax.ShapeDtypeStruct((NC, N, DW), jnp.float32),
    mesh=plsc.VectorSubcoreMesh(core_axis_name="c", subcore_axis_name="s"),
    compiler_params=pltpu.CompilerParams(use_tc_tiling_on_sc=False),
    scratch_types=[
        pltpu.VMEM_SHARED((N, DW), jnp.float32),
        pltpu.VMEM((NCH, CHUNK), jnp.int32),
        pltpu.VMEM((CHUNK, DW), jnp.float32),
    ],
)


def _agg_body(u_hbm, src_hbm, dst_hbm, parts_hbm, acc, sidx_all, didx_all,
              rows0, rows1, sem0, sem1):
    cid = lax.axis_index("c")
    sid = lax.axis_index("s")
    w = sid * NC + cid

    @pl.loop(0, CHUNK)
    def _zero(i):
        for j in range(D // 16):
            rows0[i, pl.ds(j * 16, 16)] = jnp.zeros((16,), jnp.float32)

    _zero_acc(sid, rows0, acc)
    pltpu.sync_copy(src_hbm.at[w], sidx_all)
    pltpu.sync_copy(dst_hbm.at[w], didx_all)
    plsc.subcore_barrier()

    rows = (rows0, rows1)
    sems = (sem0, sem1)

    pltpu.async_copy(u_hbm.at[sidx_all.at[0]], rows0, sem0)

    @pl.loop(0, (NCH + 1) // 2)
    def _step(k):
        c0 = 2 * k
        for b in range(2):
            c = c0 + b
            nxt = c + 1

            @pl.when(nxt < NCH)
            def _():
                pltpu.async_copy(u_hbm.at[sidx_all.at[nxt]],
                                 rows[(b + 1) % 2], sems[(b + 1) % 2])

            @pl.when(c < NCH)
            def _():
                pltpu.make_async_copy(u_hbm.at[sidx_all.at[c]],
                                      rows[b], sems[b]).wait()
                pltpu.sync_copy(rows[b], acc.at[didx_all.at[c]], add=True)

    plsc.subcore_barrier()
    _drain_acc(cid, sid, acc, parts_hbm)


_agg = pl.kernel(
    _agg_body,
    out_type=jax.ShapeDtypeStruct((NC, N, D), jnp.float32),
    mesh=plsc.VectorSubcoreMesh(core_axis_name="c", subcore_axis_name="s"),
    compiler_params=pltpu.CompilerParams(use_tc_tiling_on_sc=False),
    scratch_types=[
        pltpu.VMEM_SHARED((N, D), jnp.float32),
        pltpu.VMEM((NCH, CHUNK), jnp.int32),
        pltpu.VMEM((NCH, CHUNK), jnp.int32),
        pltpu.VMEM((CHUNK, D), jnp.float32),
        pltpu.VMEM((CHUNK, D), jnp.float32),
        pltpu.SemaphoreType.DMA,
        pltpu.SemaphoreType.DMA,
    ],
)


def _mm0_body(x_ref, degp_ref, w_ref, u_ref, dinv_ref):
    dp = degp_ref[...]
    deg = dp[0, :, 0:1] + dp[1, :, 0:1] + 1.0
    dinv = lax.rsqrt(deg)
    h = jnp.dot(x_ref[...], w_ref[...], preferred_element_type=jnp.float32)
    u_ref[...] = dinv * h
    dinv_ref[...] = dinv


_mm0 = pl.pallas_call(
    _mm0_body,
    grid=(NBLK,),
    in_specs=[
        pl.BlockSpec((R, D), lambda i: (i, 0)),
        pl.BlockSpec((NC, R, DW), lambda i: (0, i, 0)),
        pl.BlockSpec((D, D), lambda i: (0, 0)),
    ],
    out_specs=[
        pl.BlockSpec((R, D), lambda i: (i, 0)),
        pl.BlockSpec((R, 1), lambda i: (i, 0)),
    ],
    out_shape=[
        jax.ShapeDtypeStruct((N, D), jnp.float32),
        jax.ShapeDtypeStruct((N, 1), jnp.float32),
    ],
)


def _mm_body(p_ref, u_ref, dinv_ref, b_ref, w_ref, out_ref):
    p = p_ref[...]
    dinv = dinv_ref[...]
    a = jnp.maximum(dinv * (p[0] + p[1] + u_ref[...]) + b_ref[...], 0.0)
    out_ref[...] = dinv * jnp.dot(a, w_ref[...],
                                  preferred_element_type=jnp.float32)


_mm = pl.pallas_call(
    _mm_body,
    grid=(NBLK,),
    in_specs=[
        pl.BlockSpec((NC, R, D), lambda i: (0, i, 0)),
        pl.BlockSpec((R, D), lambda i: (i, 0)),
        pl.BlockSpec((R, 1), lambda i: (i, 0)),
        pl.BlockSpec((1, D), lambda i: (0, 0)),
        pl.BlockSpec((D, D), lambda i: (0, 0)),
    ],
    out_specs=pl.BlockSpec((R, D), lambda i: (i, 0)),
    out_shape=jax.ShapeDtypeStruct((N, D), jnp.float32),
)


def _pool_body(p_ref, u_ref, dinv_ref, b_ref, batch_ref, lw_ref, lb_ref,
               out_ref, pooled):
    i = pl.program_id(0)
    p = p_ref[...]
    dinv = dinv_ref[...]
    a = jnp.maximum(dinv * (p[0] + p[1] + u_ref[...]) + b_ref[...], 0.0)
    bid = batch_ref[...]

    @pl.when(i == 0)
    def _():
        pooled[...] = jnp.full((G, D), -jnp.inf, jnp.float32)

    jmin = jnp.min(bid)
    jmax = jnp.max(bid)
    for j in range(G):
        @pl.when((jmin <= j) & (j <= jmax))
        def _(j=j):
            mj = jnp.max(jnp.where(bid == j, a, -jnp.inf), axis=0,
                         keepdims=True)
            pooled[pl.ds(j, 1), :] = jnp.maximum(pooled[pl.ds(j, 1), :], mj)

    @pl.when(i == NBLK - 1)
    def _():
        out_ref[...] = (jnp.dot(pooled[...], lw_ref[...],
                                preferred_element_type=jnp.float32)
                        + lb_ref[...])


_pool = pl.pallas_call(
    _pool_body,
    grid=(NBLK,),
    in_specs=[
        pl.BlockSpec((NC, R, D), lambda i: (0, i, 0)),
        pl.BlockSpec((R, D), lambda i: (i, 0)),
        pl.BlockSpec((R, 1), lambda i: (i, 0)),
        pl.BlockSpec((1, D), lambda i: (0, 0)),
        pl.BlockSpec((R, 1), lambda i: (i, 0)),
        pl.BlockSpec((D, P), lambda i: (0, 0)),
        pl.BlockSpec((1, P), lambda i: (0, 0)),
    ],
    out_specs=pl.BlockSpec((G, P), lambda i: (0, 0)),
    out_shape=jax.ShapeDtypeStruct((G, P), jnp.float32),
    scratch_shapes=[pltpu.VMEM((G, D), jnp.float32)],
)


def kernel(x, edge_index, batch, W0, b0, W1, b1, W2, b2, lin_W, lin_b):
    src = edge_index[0].reshape(NW, NCH, CHUNK)
    dst = edge_index[1].reshape(NW, NCH, CHUNK)
    degp = _deg(dst)
    u0, dinv = _mm0(x, degp, W0)
    g0 = _agg(u0, src, dst)
    u1 = _mm(g0, u0, dinv, b0.reshape(1, D), W1)
    g1 = _agg(u1, src, dst)
    u2 = _mm(g1, u1, dinv, b1.reshape(1, D), W2)
    g2 = _agg(u2, src, dst)
    out = _pool(g2, u2, dinv, b2.reshape(1, D), batch.reshape(N, 1),
                lin_W, lin_b.reshape(1, P))
    return out


# speedup vs baseline: 26.0363x; 1.0010x over previous
import jax
import jax.numpy as jnp
from jax import lax
from jax.experimental import pallas as pl
from jax.experimental.pallas import tpu as pltpu
from jax.experimental.pallas import tpu_sc as plsc

N = 10000
E = 320000
D = 128
G = 64
P = 16

NC = 2
NS = 16
NW = NC * NS
EPW = E // NW
CHUNK = 80
NCH = EPW // CHUNK
SPLIT = 624
LAST = N - SPLIT * (NS - 1)

R = 1000
NBLK = N // R


def _zero_acc(sid, zbuf, acc):
    @pl.loop(0, pl.cdiv(N // CHUNK, NS))
    def _(k):
        c = k * NS + sid

        @pl.when(c < N // CHUNK)
        def _():
            pltpu.sync_copy(zbuf, acc.at[pl.ds(c * CHUNK, CHUNK)])


def _drain_acc(cid, sid, acc, parts_hbm):
    @pl.when(sid < NS - 1)
    def _():
        pltpu.sync_copy(acc.at[pl.ds(sid * SPLIT, SPLIT)],
                        parts_hbm.at[cid, pl.ds(sid * SPLIT, SPLIT)])

    @pl.when(sid == NS - 1)
    def _():
        pltpu.sync_copy(acc.at[pl.ds((NS - 1) * SPLIT, LAST)],
                        parts_hbm.at[cid, pl.ds((NS - 1) * SPLIT, LAST)])


DW = 16


def _deg_body(dst_hbm, parts_hbm, acc, didx_all, ones_v):
    cid = lax.axis_index("c")
    sid = lax.axis_index("s")
    w = sid * NC + cid

    @pl.loop(0, CHUNK)
    def _zero(i):
        ones_v[i] = jnp.zeros((DW,), jnp.float32)

    _zero_acc(sid, ones_v, acc)
    pltpu.sync_copy(dst_hbm.at[w], didx_all)

    @pl.loop(0, CHUNK)
    def _fill(i):
        ones_v[i] = jnp.full((DW,), 1.0, jnp.float32)

    plsc.subcore_barrier()

    @pl.loop(0, NCH)
    def _step(it):
        pltpu.sync_copy(ones_v, acc.at[didx_all.at[it]], add=True)

    plsc.subcore_barrier()
    _drain_acc(cid, sid, acc, parts_hbm)


_deg = pl.kernel(
    _deg_body,
    out_type=jax.ShapeDtypeStruct((NC, N, DW), jnp.float32),
    mesh=plsc.VectorSubcoreMesh(core_axis_name="c", subcore_axis_name="s"),
    compiler_params=pltpu.CompilerParams(use_tc_tiling_on_sc=False),
    scratch_types=[
        pltpu.VMEM_SHARED((N, DW), jnp.float32),
        pltpu.VMEM((NCH, CHUNK), jnp.int32),
        pltpu.VMEM((CHUNK, DW), jnp.float32),
    ],
)


def _agg_body(u_hbm, src_hbm, dst_hbm, parts_hbm, acc, sidx_all, didx_all,
              rows0, rows1, sem0, sem1):
    cid = lax.axis_index("c")
    sid = lax.axis_index("s")
    w = sid * NC + cid

    @pl.loop(0, CHUNK)
    def _zero(i):
        for j in range(D // 16):
            rows0[i, pl.ds(j * 16, 16)] = jnp.zeros((16,), jnp.float32)

    _zero_acc(sid, rows0, acc)
    pltpu.sync_copy(src_hbm.at[w], sidx_all)
    pltpu.sync_copy(dst_hbm.at[w], didx_all)
    plsc.subcore_barrier()

    rows = (rows0, rows1)
    sems = (sem0, sem1)

    pltpu.async_copy(u_hbm.at[sidx_all.at[0]], rows0, sem0)

    @pl.loop(0, (NCH + 1) // 2)
    def _step(k):
        c0 = 2 * k
        for b in range(2):
            c = c0 + b
            nxt = c + 1

            @pl.when(nxt < NCH)
            def _():
                pltpu.async_copy(u_hbm.at[sidx_all.at[nxt]],
                                 rows[(b + 1) % 2], sems[(b + 1) % 2])

            @pl.when(c < NCH)
            def _():
                pltpu.make_async_copy(u_hbm.at[sidx_all.at[c]],
                                      rows[b], sems[b]).wait()
                pltpu.sync_copy(rows[b], acc.at[didx_all.at[c]], add=True)

    plsc.subcore_barrier()
    _drain_acc(cid, sid, acc, parts_hbm)


_agg = pl.kernel(
    _agg_body,
    out_type=jax.ShapeDtypeStruct((NC, N, D), jnp.float32),
    mesh=plsc.VectorSubcoreMesh(core_axis_name="c", subcore_axis_name="s"),
    compiler_params=pltpu.CompilerParams(use_tc_tiling_on_sc=False),
    scratch_types=[
        pltpu.VMEM_SHARED((N, D), jnp.float32),
        pltpu.VMEM((NCH, CHUNK), jnp.int32),
        pltpu.VMEM((NCH, CHUNK), jnp.int32),
        pltpu.VMEM((CHUNK, D), jnp.float32),
        pltpu.VMEM((CHUNK, D), jnp.float32),
        pltpu.SemaphoreType.DMA,
        pltpu.SemaphoreType.DMA,
    ],
)


def _h0_body(x_ref, w_ref, h_ref):
    h_ref[...] = jnp.dot(x_ref[...], w_ref[...],
                         preferred_element_type=jnp.float32)


_h0 = pl.pallas_call(
    _h0_body,
    grid=(NBLK,),
    in_specs=[
        pl.BlockSpec((R, D), lambda i: (i, 0)),
        pl.BlockSpec((D, D), lambda i: (0, 0)),
    ],
    out_specs=pl.BlockSpec((R, D), lambda i: (i, 0)),
    out_shape=jax.ShapeDtypeStruct((N, D), jnp.float32),
)


def _scale0_body(h_ref, degp_ref, u_ref, dinv_ref):
    dp = degp_ref[...]
    deg = dp[0, :, 0:1] + dp[1, :, 0:1] + 1.0
    dinv = lax.rsqrt(deg)
    u_ref[...] = dinv * h_ref[...]
    dinv_ref[...] = dinv


_scale0 = pl.pallas_call(
    _scale0_body,
    grid=(NBLK,),
    in_specs=[
        pl.BlockSpec((R, D), lambda i: (i, 0)),
        pl.BlockSpec((NC, R, DW), lambda i: (0, i, 0)),
    ],
    out_specs=[
        pl.BlockSpec((R, D), lambda i: (i, 0)),
        pl.BlockSpec((R, 1), lambda i: (i, 0)),
    ],
    out_shape=[
        jax.ShapeDtypeStruct((N, D), jnp.float32),
        jax.ShapeDtypeStruct((N, 1), jnp.float32),
    ],
)


def _mm_body(p_ref, u_ref, dinv_ref, b_ref, w_ref, out_ref):
    p = p_ref[...]
    dinv = dinv_ref[...]
    a = jnp.maximum(dinv * (p[0] + p[1] + u_ref[...]) + b_ref[...], 0.0)
    out_ref[...] = dinv * jnp.dot(a, w_ref[...],
                                  preferred_element_type=jnp.float32)


_mm = pl.pallas_call(
    _mm_body,
    grid=(NBLK,),
    in_specs=[
        pl.BlockSpec((NC, R, D), lambda i: (0, i, 0)),
        pl.BlockSpec((R, D), lambda i: (i, 0)),
        pl.BlockSpec((R, 1), lambda i: (i, 0)),
        pl.BlockSpec((1, D), lambda i: (0, 0)),
        pl.BlockSpec((D, D), lambda i: (0, 0)),
    ],
    out_specs=pl.BlockSpec((R, D), lambda i: (i, 0)),
    out_shape=jax.ShapeDtypeStruct((N, D), jnp.float32),
)


def _pool_body(p_ref, u_ref, dinv_ref, b_ref, batch_ref, lw_ref, lb_ref,
               out_ref, pooled):
    i = pl.program_id(0)
    p = p_ref[...]
    dinv = dinv_ref[...]
    a = jnp.maximum(dinv * (p[0] + p[1] + u_ref[...]) + b_ref[...], 0.0)
    bid = batch_ref[...]

    @pl.when(i == 0)
    def _():
        pooled[...] = jnp.full((G, D), -jnp.inf, jnp.float32)

    jmin = jnp.min(bid)
    jmax = jnp.max(bid)
    for j in range(G):
        @pl.when((jmin <= j) & (j <= jmax))
        def _(j=j):
            mj = jnp.max(jnp.where(bid == j, a, -jnp.inf), axis=0,
                         keepdims=True)
            pooled[pl.ds(j, 1), :] = jnp.maximum(pooled[pl.ds(j, 1), :], mj)

    @pl.when(i == NBLK - 1)
    def _():
        out_ref[...] = (jnp.dot(pooled[...], lw_ref[...],
                                preferred_element_type=jnp.float32)
                        + lb_ref[...])


_pool = pl.pallas_call(
    _pool_body,
    grid=(NBLK,),
    in_specs=[
        pl.BlockSpec((NC, R, D), lambda i: (0, i, 0)),
        pl.BlockSpec((R, D), lambda i: (i, 0)),
        pl.BlockSpec((R, 1), lambda i: (i, 0)),
        pl.BlockSpec((1, D), lambda i: (0, 0)),
        pl.BlockSpec((R, 1), lambda i: (i, 0)),
        pl.BlockSpec((D, P), lambda i: (0, 0)),
        pl.BlockSpec((1, P), lambda i: (0, 0)),
    ],
    out_specs=pl.BlockSpec((G, P), lambda i: (0, 0)),
    out_shape=jax.ShapeDtypeStruct((G, P), jnp.float32),
    scratch_shapes=[pltpu.VMEM((G, D), jnp.float32)],
)


def kernel(x, edge_index, batch, W0, b0, W1, b1, W2, b2, lin_W, lin_b):
    src = edge_index[0].reshape(NW, NCH, CHUNK)
    dst = edge_index[1].reshape(NW, NCH, CHUNK)
    degp = _deg(dst)
    h0 = _h0(x, W0)
    u0, dinv = _scale0(h0, degp)
    g0 = _agg(u0, src, dst)
    u1 = _mm(g0, u0, dinv, b0.reshape(1, D), W1)
    g1 = _agg(u1, src, dst)
    u2 = _mm(g1, u1, dinv, b1.reshape(1, D), W2)
    g2 = _agg(u2, src, dst)
    out = _pool(g2, u2, dinv, b2.reshape(1, D), batch.reshape(N, 1),
                lin_W, lin_b.reshape(1, P))
    return out


# speedup vs baseline: 30.2832x; 1.1631x over previous
import jax
import jax.numpy as jnp
from jax import lax
from jax.experimental import pallas as pl
from jax.experimental.pallas import tpu as pltpu
from jax.experimental.pallas import tpu_sc as plsc

N = 10000
E = 320000
D = 128
G = 64
P = 16

NC = 2
NS = 16
NW = NC * NS
EPW = E // NW
CHUNK = 80
NCH = EPW // CHUNK
SPLIT = 624
LAST = N - SPLIT * (NS - 1)

R = 1000
NBLK = N // R


def _zero_acc(sid, zbuf, acc):
    @pl.loop(0, pl.cdiv(N // CHUNK, NS))
    def _(k):
        c = k * NS + sid

        @pl.when(c < N // CHUNK)
        def _():
            pltpu.sync_copy(zbuf, acc.at[pl.ds(c * CHUNK, CHUNK)])


def _drain_acc(cid, sid, acc, parts_hbm):
    @pl.when(sid < NS - 1)
    def _():
        pltpu.sync_copy(acc.at[pl.ds(sid * SPLIT, SPLIT)],
                        parts_hbm.at[cid, pl.ds(sid * SPLIT, SPLIT)])

    @pl.when(sid == NS - 1)
    def _():
        pltpu.sync_copy(acc.at[pl.ds((NS - 1) * SPLIT, LAST)],
                        parts_hbm.at[cid, pl.ds((NS - 1) * SPLIT, LAST)])


DW = 16


def _deg_body(dst_hbm, parts_hbm, acc, didx_all, ones_v):
    cid = lax.axis_index("c")
    sid = lax.axis_index("s")
    w = sid * NC + cid

    @pl.loop(0, CHUNK)
    def _zero(i):
        ones_v[i] = jnp.zeros((DW,), jnp.float32)

    _zero_acc(sid, ones_v, acc)
    pltpu.sync_copy(dst_hbm.at[w], didx_all)

    @pl.loop(0, CHUNK)
    def _fill(i):
        ones_v[i] = jnp.full((DW,), 1.0, jnp.float32)

    plsc.subcore_barrier()

    @pl.loop(0, NCH)
    def _step(it):
        pltpu.sync_copy(ones_v, acc.at[didx_all.at[it]], add=True)

    plsc.subcore_barrier()
    _drain_acc(cid, sid, acc, parts_hbm)


_deg = pl.kernel(
    _deg_body,
    out_type=jax.ShapeDtypeStruct((NC, N, DW), jnp.float32),
    mesh=plsc.VectorSubcoreMesh(core_axis_name="c", subcore_axis_name="s"),
    compiler_params=pltpu.CompilerParams(use_tc_tiling_on_sc=False),
    scratch_types=[
        pltpu.VMEM_SHARED((N, DW), jnp.float32),
        pltpu.VMEM((NCH, CHUNK), jnp.int32),
        pltpu.VMEM((CHUNK, DW), jnp.float32),
    ],
)


NBUF = 3


def _agg_body(u_hbm, src_hbm, dst_hbm, parts_hbm, acc, sidx_all, didx_all,
              rows0, rows1, rows2, sem0, sem1, sem2):
    cid = lax.axis_index("c")
    sid = lax.axis_index("s")
    w = sid * NC + cid

    @pl.loop(0, CHUNK)
    def _zero(i):
        for j in range(D // 16):
            rows0[i, pl.ds(j * 16, 16)] = jnp.zeros((16,), jnp.float32)

    _zero_acc(sid, rows0, acc)
    pltpu.sync_copy(src_hbm.at[w], sidx_all)
    pltpu.sync_copy(dst_hbm.at[w], didx_all)
    plsc.subcore_barrier()

    rows = (rows0, rows1, rows2)
    sems = (sem0, sem1, sem2)

    for c in range(NBUF - 1):
        pltpu.async_copy(u_hbm.at[sidx_all.at[c]], rows[c], sems[c])

    @pl.loop(0, (NCH + NBUF - 1) // NBUF)
    def _step(k):
        c0 = NBUF * k
        for b in range(NBUF):
            c = c0 + b
            nxt = c + NBUF - 1
            bn = (b + NBUF - 1) % NBUF

            @pl.when(nxt < NCH)
            def _():
                pltpu.async_copy(u_hbm.at[sidx_all.at[nxt]],
                                 rows[bn], sems[bn])

            @pl.when(c < NCH)
            def _():
                pltpu.make_async_copy(u_hbm.at[sidx_all.at[c]],
                                      rows[b], sems[b]).wait()
                pltpu.sync_copy(rows[b], acc.at[didx_all.at[c]],
                                add=True)

    plsc.subcore_barrier()
    _drain_acc(cid, sid, acc, parts_hbm)


_agg = pl.kernel(
    _agg_body,
    out_type=jax.ShapeDtypeStruct((NC, N, D), jnp.float32),
    mesh=plsc.VectorSubcoreMesh(core_axis_name="c", subcore_axis_name="s"),
    compiler_params=pltpu.CompilerParams(use_tc_tiling_on_sc=False),
    scratch_types=[
        pltpu.VMEM_SHARED((N, D), jnp.float32),
        pltpu.VMEM((NCH, CHUNK), jnp.int32),
        pltpu.VMEM((NCH, CHUNK), jnp.int32),
        pltpu.VMEM((CHUNK, D), jnp.float32),
        pltpu.VMEM((CHUNK, D), jnp.float32),
        pltpu.VMEM((CHUNK, D), jnp.float32),
        pltpu.SemaphoreType.DMA,
        pltpu.SemaphoreType.DMA,
        pltpu.SemaphoreType.DMA,
    ],
)


def _h0_body(x_ref, w_ref, h_ref):
    h_ref[...] = jnp.dot(x_ref[...], w_ref[...],
                         preferred_element_type=jnp.float32)


_h0 = pl.pallas_call(
    _h0_body,
    grid=(NBLK,),
    in_specs=[
        pl.BlockSpec((R, D), lambda i: (i, 0)),
        pl.BlockSpec((D, D), lambda i: (0, 0)),
    ],
    out_specs=pl.BlockSpec((R, D), lambda i: (i, 0)),
    out_shape=jax.ShapeDtypeStruct((N, D), jnp.float32),
)


def _scale0_body(h_ref, degp_ref, u_ref, dinv_ref):
    dp = degp_ref[...]
    deg = dp[0, :, 0:1] + dp[1, :, 0:1] + 1.0
    dinv = lax.rsqrt(deg)
    u_ref[...] = dinv * h_ref[...]
    dinv_ref[...] = dinv


_scale0 = pl.pallas_call(
    _scale0_body,
    grid=(NBLK,),
    in_specs=[
        pl.BlockSpec((R, D), lambda i: (i, 0)),
        pl.BlockSpec((NC, R, DW), lambda i: (0, i, 0)),
    ],
    out_specs=[
        pl.BlockSpec((R, D), lambda i: (i, 0)),
        pl.BlockSpec((R, 1), lambda i: (i, 0)),
    ],
    out_shape=[
        jax.ShapeDtypeStruct((N, D), jnp.float32),
        jax.ShapeDtypeStruct((N, 1), jnp.float32),
    ],
)


def _mm_body(p_ref, u_ref, dinv_ref, b_ref, w_ref, out_ref):
    p = p_ref[...]
    dinv = dinv_ref[...]
    a = jnp.maximum(dinv * (p[0] + p[1] + u_ref[...]) + b_ref[...], 0.0)
    out_ref[...] = dinv * jnp.dot(a, w_ref[...],
                                  preferred_element_type=jnp.float32)


_mm = pl.pallas_call(
    _mm_body,
    grid=(NBLK,),
    in_specs=[
        pl.BlockSpec((NC, R, D), lambda i: (0, i, 0)),
        pl.BlockSpec((R, D), lambda i: (i, 0)),
        pl.BlockSpec((R, 1), lambda i: (i, 0)),
        pl.BlockSpec((1, D), lambda i: (0, 0)),
        pl.BlockSpec((D, D), lambda i: (0, 0)),
    ],
    out_specs=pl.BlockSpec((R, D), lambda i: (i, 0)),
    out_shape=jax.ShapeDtypeStruct((N, D), jnp.float32),
)


def _pool_body(p_ref, u_ref, dinv_ref, b_ref, batch_ref, lw_ref, lb_ref,
               out_ref, pooled):
    i = pl.program_id(0)
    p = p_ref[...]
    dinv = dinv_ref[...]
    a = jnp.maximum(dinv * (p[0] + p[1] + u_ref[...]) + b_ref[...], 0.0)
    bid = batch_ref[...]

    @pl.when(i == 0)
    def _():
        pooled[...] = jnp.full((G, D), -jnp.inf, jnp.float32)

    jmin = jnp.min(bid)
    jmax = jnp.max(bid)
    for j in range(G):
        @pl.when((jmin <= j) & (j <= jmax))
        def _(j=j):
            mj = jnp.max(jnp.where(bid == j, a, -jnp.inf), axis=0,
                         keepdims=True)
            pooled[pl.ds(j, 1), :] = jnp.maximum(pooled[pl.ds(j, 1), :], mj)

    @pl.when(i == NBLK - 1)
    def _():
        out_ref[...] = (jnp.dot(pooled[...], lw_ref[...],
                                preferred_element_type=jnp.float32)
                        + lb_ref[...])


_pool = pl.pallas_call(
    _pool_body,
    grid=(NBLK,),
    in_specs=[
        pl.BlockSpec((NC, R, D), lambda i: (0, i, 0)),
        pl.BlockSpec((R, D), lambda i: (i, 0)),
        pl.BlockSpec((R, 1), lambda i: (i, 0)),
        pl.BlockSpec((1, D), lambda i: (0, 0)),
        pl.BlockSpec((R, 1), lambda i: (i, 0)),
        pl.BlockSpec((D, P), lambda i: (0, 0)),
        pl.BlockSpec((1, P), lambda i: (0, 0)),
    ],
    out_specs=pl.BlockSpec((G, P), lambda i: (0, 0)),
    out_shape=jax.ShapeDtypeStruct((G, P), jnp.float32),
    scratch_shapes=[pltpu.VMEM((G, D), jnp.float32)],
)


def kernel(x, edge_index, batch, W0, b0, W1, b1, W2, b2, lin_W, lin_b):
    src = edge_index[0].reshape(NW, NCH, CHUNK)
    dst = edge_index[1].reshape(NW, NCH, CHUNK)
    degp = _deg(dst)
    h0 = _h0(x, W0)
    u0, dinv = _scale0(h0, degp)
    g0 = _agg(u0, src, dst)
    u1 = _mm(g0, u0, dinv, b0.reshape(1, D), W1)
    g1 = _agg(u1, src, dst)
    u2 = _mm(g1, u1, dinv, b1.reshape(1, D), W2)
    g2 = _agg(u2, src, dst)
    out = _pool(g2, u2, dinv, b2.reshape(1, D), batch.reshape(N, 1),
                lin_W, lin_b.reshape(1, P))
    return out


# speedup vs baseline: 30.3082x; 1.0008x over previous
import jax
import jax.numpy as jnp
from jax import lax
from jax.experimental import pallas as pl
from jax.experimental.pallas import tpu as pltpu
from jax.experimental.pallas import tpu_sc as plsc

N = 10000
E = 320000
D = 128
G = 64
P = 16

NC = 2
NS = 16
NW = NC * NS
EPW = E // NW
CHUNK = 80
NCH = EPW // CHUNK
SPLIT = 624
LAST = N - SPLIT * (NS - 1)

R = 1000
NBLK = N // R


def _zero_acc(sid, zbuf, acc):
    @pl.loop(0, pl.cdiv(N // CHUNK, NS))
    def _(k):
        c = k * NS + sid

        @pl.when(c < N // CHUNK)
        def _():
            pltpu.sync_copy(zbuf, acc.at[pl.ds(c * CHUNK, CHUNK)])


def _drain_acc(cid, sid, acc, parts_hbm):
    @pl.when(sid < NS - 1)
    def _():
        pltpu.sync_copy(acc.at[pl.ds(sid * SPLIT, SPLIT)],
                        parts_hbm.at[cid, pl.ds(sid * SPLIT, SPLIT)])

    @pl.when(sid == NS - 1)
    def _():
        pltpu.sync_copy(acc.at[pl.ds((NS - 1) * SPLIT, LAST)],
                        parts_hbm.at[cid, pl.ds((NS - 1) * SPLIT, LAST)])


DW = 16


def _deg_body(dst_hbm, parts_hbm, acc, didx_all, ones_v):
    cid = lax.axis_index("c")
    sid = lax.axis_index("s")
    w = sid * NC + cid

    @pl.loop(0, CHUNK)
    def _zero(i):
        ones_v[i] = jnp.zeros((DW,), jnp.float32)

    _zero_acc(sid, ones_v, acc)
    pltpu.sync_copy(dst_hbm.at[w], didx_all)

    @pl.loop(0, CHUNK)
    def _fill(i):
        ones_v[i] = jnp.full((DW,), 1.0, jnp.float32)

    plsc.subcore_barrier()

    @pl.loop(0, NCH)
    def _step(it):
        pltpu.sync_copy(ones_v, acc.at[didx_all.at[it]], add=True)

    plsc.subcore_barrier()
    _drain_acc(cid, sid, acc, parts_hbm)


_deg = pl.kernel(
    _deg_body,
    out_type=jax.ShapeDtypeStruct((NC, N, DW), jnp.float32),
    mesh=plsc.VectorSubcoreMesh(core_axis_name="c", subcore_axis_name="s"),
    compiler_params=pltpu.CompilerParams(use_tc_tiling_on_sc=False),
    scratch_types=[
        pltpu.VMEM_SHARED((N, DW), jnp.float32),
        pltpu.VMEM((NCH, CHUNK), jnp.int32),
        pltpu.VMEM((CHUNK, DW), jnp.float32),
    ],
)


NBUF = 3


def _agg_body(u_hbm, src_hbm, dst_hbm, parts_hbm, acc, sidx_all, didx_all,
              rows0, rows1, rows2, sem0, sem1, sem2):
    cid = lax.axis_index("c")
    sid = lax.axis_index("s")
    w = sid * NC + cid

    @pl.loop(0, CHUNK)
    def _zero(i):
        for j in range(D // 16):
            rows0[i, pl.ds(j * 16, 16)] = jnp.zeros((16,), jnp.float32)

    _zero_acc(sid, rows0, acc)
    pltpu.sync_copy(src_hbm.at[w], sidx_all)
    pltpu.sync_copy(dst_hbm.at[w], didx_all)
    plsc.subcore_barrier()

    rows = (rows0, rows1, rows2)
    sems = (sem0, sem1, sem2)

    for c in range(NBUF - 1):
        pltpu.async_copy(u_hbm.at[sidx_all.at[c]], rows[c], sems[c])

    @pl.loop(0, (NCH + NBUF - 1) // NBUF)
    def _step(k):
        c0 = NBUF * k
        for b in range(NBUF):
            c = c0 + b
            nxt = c + NBUF - 1
            bn = (b + NBUF - 1) % NBUF

            @pl.when(nxt < NCH)
            def _():
                pltpu.async_copy(u_hbm.at[sidx_all.at[nxt]],
                                 rows[bn], sems[bn])

            @pl.when(c < NCH)
            def _():
                pltpu.make_async_copy(u_hbm.at[sidx_all.at[c]],
                                      rows[b], sems[b]).wait()
                pltpu.sync_copy(rows[b], acc.at[didx_all.at[c]],
                                add=True)

    plsc.subcore_barrier()
    _drain_acc(cid, sid, acc, parts_hbm)


_agg = pl.kernel(
    _agg_body,
    out_type=jax.ShapeDtypeStruct((NC, N, D), jnp.float32),
    mesh=plsc.VectorSubcoreMesh(core_axis_name="c", subcore_axis_name="s"),
    compiler_params=pltpu.CompilerParams(use_tc_tiling_on_sc=False),
    scratch_types=[
        pltpu.VMEM_SHARED((N, D), jnp.float32),
        pltpu.VMEM((NCH, CHUNK), jnp.int32),
        pltpu.VMEM((NCH, CHUNK), jnp.int32),
        pltpu.VMEM((CHUNK, D), jnp.float32),
        pltpu.VMEM((CHUNK, D), jnp.float32),
        pltpu.VMEM((CHUNK, D), jnp.float32),
        pltpu.SemaphoreType.DMA,
        pltpu.SemaphoreType.DMA,
        pltpu.SemaphoreType.DMA,
    ],
)


def _mm0_body(x_ref, degp_ref, w_ref, u_ref, dinv_ref):
    dp = degp_ref[...]
    deg = dp[0, :, 0:1] + dp[1, :, 0:1] + 1.0
    dinv = lax.rsqrt(deg)
    h = jnp.dot(x_ref[...], w_ref[...], preferred_element_type=jnp.float32)
    u_ref[...] = dinv * h
    dinv_ref[...] = dinv


_mm0 = pl.pallas_call(
    _mm0_body,
    grid=(NBLK,),
    in_specs=[
        pl.BlockSpec((R, D), lambda i: (i, 0)),
        pl.BlockSpec((NC, R, DW), lambda i: (0, i, 0)),
        pl.BlockSpec((D, D), lambda i: (0, 0)),
    ],
    out_specs=[
        pl.BlockSpec((R, D), lambda i: (i, 0)),
        pl.BlockSpec((R, 1), lambda i: (i, 0)),
    ],
    out_shape=[
        jax.ShapeDtypeStruct((N, D), jnp.float32),
        jax.ShapeDtypeStruct((N, 1), jnp.float32),
    ],
)


def _mm_body(p_ref, u_ref, dinv_ref, b_ref, w_ref, out_ref):
    p = p_ref[...]
    dinv = dinv_ref[...]
    a = jnp.maximum(dinv * (p[0] + p[1] + u_ref[...]) + b_ref[...], 0.0)
    out_ref[...] = dinv * jnp.dot(a, w_ref[...],
                                  preferred_element_type=jnp.float32)


_mm = pl.pallas_call(
    _mm_body,
    grid=(NBLK,),
    in_specs=[
        pl.BlockSpec((NC, R, D), lambda i: (0, i, 0)),
        pl.BlockSpec((R, D), lambda i: (i, 0)),
        pl.BlockSpec((R, 1), lambda i: (i, 0)),
        pl.BlockSpec((1, D), lambda i: (0, 0)),
        pl.BlockSpec((D, D), lambda i: (0, 0)),
    ],
    out_specs=pl.BlockSpec((R, D), lambda i: (i, 0)),
    out_shape=jax.ShapeDtypeStruct((N, D), jnp.float32),
)


def _pool_body(p_ref, u_ref, dinv_ref, b_ref, batch_ref, lw_ref, lb_ref,
               out_ref, pooled):
    i = pl.program_id(0)
    p = p_ref[...]
    dinv = dinv_ref[...]
    a = jnp.maximum(dinv * (p[0] + p[1] + u_ref[...]) + b_ref[...], 0.0)
    bid = batch_ref[...]

    @pl.when(i == 0)
    def _():
        pooled[...] = jnp.full((G, D), -jnp.inf, jnp.float32)

    jmin = jnp.min(bid)
    jmax = jnp.max(bid)
    for j in range(G):
        @pl.when((jmin <= j) & (j <= jmax))
        def _(j=j):
            mj = jnp.max(jnp.where(bid == j, a, -jnp.inf), axis=0,
                         keepdims=True)
            pooled[pl.ds(j, 1), :] = jnp.maximum(pooled[pl.ds(j, 1), :], mj)

    @pl.when(i == NBLK - 1)
    def _():
        out_ref[...] = (jnp.dot(pooled[...], lw_ref[...],
                                preferred_element_type=jnp.float32)
                        + lb_ref[...])


_pool = pl.pallas_call(
    _pool_body,
    grid=(NBLK,),
    in_specs=[
        pl.BlockSpec((NC, R, D), lambda i: (0, i, 0)),
        pl.BlockSpec((R, D), lambda i: (i, 0)),
        pl.BlockSpec((R, 1), lambda i: (i, 0)),
        pl.BlockSpec((1, D), lambda i: (0, 0)),
        pl.BlockSpec((R, 1), lambda i: (i, 0)),
        pl.BlockSpec((D, P), lambda i: (0, 0)),
        pl.BlockSpec((1, P), lambda i: (0, 0)),
    ],
    out_specs=pl.BlockSpec((G, P), lambda i: (0, 0)),
    out_shape=jax.ShapeDtypeStruct((G, P), jnp.float32),
    scratch_shapes=[pltpu.VMEM((G, D), jnp.float32)],
)


def kernel(x, edge_index, batch, W0, b0, W1, b1, W2, b2, lin_W, lin_b):
    src = edge_index[0].reshape(NW, NCH, CHUNK)
    dst = edge_index[1].reshape(NW, NCH, CHUNK)
    degp = _deg(dst)
    u0, dinv = _mm0(x, degp, W0)
    g0 = _agg(u0, src, dst)
    u1 = _mm(g0, u0, dinv, b0.reshape(1, D), W1)
    g1 = _agg(u1, src, dst)
    u2 = _mm(g1, u1, dinv, b1.reshape(1, D), W2)
    g2 = _agg(u2, src, dst)
    out = _pool(g2, u2, dinv, b2.reshape(1, D), batch.reshape(N, 1),
                lin_W, lin_b.reshape(1, P))
    return out


# speedup vs baseline: 30.7876x; 1.0158x over previous
import jax
import jax.numpy as jnp
from jax import lax
from jax.experimental import pallas as pl
from jax.experimental.pallas import tpu as pltpu
from jax.experimental.pallas import tpu_sc as plsc

N = 10000
E = 320000
D = 128
G = 64
P = 16

NC = 2
NS = 16
NW = NC * NS
EPW = E // NW
CHUNK = 80
NCH = EPW // CHUNK
SPLIT = 624
LAST = N - SPLIT * (NS - 1)

R = 1000
NBLK = N // R


def _zero_acc(sid, zbuf, acc):
    @pl.loop(0, pl.cdiv(N // CHUNK, NS))
    def _(k):
        c = k * NS + sid

        @pl.when(c < N // CHUNK)
        def _():
            pltpu.sync_copy(zbuf, acc.at[pl.ds(c * CHUNK, CHUNK)])


def _drain_acc(cid, sid, acc, parts_hbm):
    @pl.when(sid < NS - 1)
    def _():
        pltpu.sync_copy(acc.at[pl.ds(sid * SPLIT, SPLIT)],
                        parts_hbm.at[cid, pl.ds(sid * SPLIT, SPLIT)])

    @pl.when(sid == NS - 1)
    def _():
        pltpu.sync_copy(acc.at[pl.ds((NS - 1) * SPLIT, LAST)],
                        parts_hbm.at[cid, pl.ds((NS - 1) * SPLIT, LAST)])


DW = 16


def _deg_body(dst_hbm, parts_hbm, acc, didx_all, ones_v):
    cid = lax.axis_index("c")
    sid = lax.axis_index("s")
    w = sid * NC + cid

    @pl.loop(0, CHUNK)
    def _zero(i):
        ones_v[i] = jnp.zeros((DW,), jnp.float32)

    _zero_acc(sid, ones_v, acc)
    pltpu.sync_copy(dst_hbm.at[w], didx_all)

    @pl.loop(0, CHUNK)
    def _fill(i):
        ones_v[i] = jnp.full((DW,), 1.0, jnp.float32)

    plsc.subcore_barrier()

    @pl.loop(0, NCH)
    def _step(it):
        pltpu.sync_copy(ones_v, acc.at[didx_all.at[it]], add=True)

    plsc.subcore_barrier()
    _drain_acc(cid, sid, acc, parts_hbm)


_deg = pl.kernel(
    _deg_body,
    out_type=jax.ShapeDtypeStruct((NC, N, DW), jnp.float32),
    mesh=plsc.VectorSubcoreMesh(core_axis_name="c", subcore_axis_name="s"),
    compiler_params=pltpu.CompilerParams(use_tc_tiling_on_sc=False),
    scratch_types=[
        pltpu.VMEM_SHARED((N, DW), jnp.float32),
        pltpu.VMEM((NCH, CHUNK), jnp.int32),
        pltpu.VMEM((CHUNK, DW), jnp.float32),
    ],
)


NBUF = 3


def _agg_body(u_hbm, src_hbm, dst_hbm, parts_hbm, acc, sidx_all, didx_all,
              rows0, rows1, rows2, sem0, sem1, sem2):
    cid = lax.axis_index("c")
    sid = lax.axis_index("s")
    w = sid * NC + cid

    cps = pltpu.async_copy(src_hbm.at[w], sidx_all, sem0)
    cpd = pltpu.async_copy(dst_hbm.at[w], didx_all, sem1)

    @pl.loop(0, CHUNK)
    def _zero(i):
        for j in range(D // 16):
            rows0[i, pl.ds(j * 16, 16)] = jnp.zeros((16,), jnp.float32)

    _zero_acc(sid, rows0, acc)
    cps.wait()
    cpd.wait()
    plsc.subcore_barrier()

    rows = (rows0, rows1, rows2)
    sems = (sem0, sem1, sem2)

    for c in range(NBUF - 1):
        pltpu.async_copy(u_hbm.at[sidx_all.at[c]], rows[c], sems[c])

    @pl.loop(0, (NCH + NBUF - 1) // NBUF)
    def _step(k):
        c0 = NBUF * k
        for b in range(NBUF):
            c = c0 + b
            nxt = c + NBUF - 1
            bn = (b + NBUF - 1) % NBUF

            @pl.when(nxt < NCH)
            def _():
                pltpu.async_copy(u_hbm.at[sidx_all.at[nxt]],
                                 rows[bn], sems[bn])

            @pl.when(c < NCH)
            def _():
                pltpu.make_async_copy(u_hbm.at[sidx_all.at[c]],
                                      rows[b], sems[b]).wait()
                pltpu.sync_copy(rows[b], acc.at[didx_all.at[c]],
                                add=True)

    plsc.subcore_barrier()
    _drain_acc(cid, sid, acc, parts_hbm)


_agg = pl.kernel(
    _agg_body,
    out_type=jax.ShapeDtypeStruct((NC, N, D), jnp.float32),
    mesh=plsc.VectorSubcoreMesh(core_axis_name="c", subcore_axis_name="s"),
    compiler_params=pltpu.CompilerParams(use_tc_tiling_on_sc=False),
    scratch_types=[
        pltpu.VMEM_SHARED((N, D), jnp.float32),
        pltpu.VMEM((NCH, CHUNK), jnp.int32),
        pltpu.VMEM((NCH, CHUNK), jnp.int32),
        pltpu.VMEM((CHUNK, D), jnp.float32),
        pltpu.VMEM((CHUNK, D), jnp.float32),
        pltpu.VMEM((CHUNK, D), jnp.float32),
        pltpu.SemaphoreType.DMA,
        pltpu.SemaphoreType.DMA,
        pltpu.SemaphoreType.DMA,
    ],
)


def _mm0_body(x_ref, degp_ref, w_ref, u_ref, dinv_ref):
    dp = degp_ref[...]
    deg = dp[0, :, 0:1] + dp[1, :, 0:1] + 1.0
    dinv = lax.rsqrt(deg)
    h = jnp.dot(x_ref[...], w_ref[...], preferred_element_type=jnp.float32)
    u_ref[...] = dinv * h
    dinv_ref[...] = dinv


_mm0 = pl.pallas_call(
    _mm0_body,
    grid=(NBLK,),
    in_specs=[
        pl.BlockSpec((R, D), lambda i: (i, 0)),
        pl.BlockSpec((NC, R, DW), lambda i: (0, i, 0)),
        pl.BlockSpec((D, D), lambda i: (0, 0)),
    ],
    out_specs=[
        pl.BlockSpec((R, D), lambda i: (i, 0)),
        pl.BlockSpec((R, 1), lambda i: (i, 0)),
    ],
    out_shape=[
        jax.ShapeDtypeStruct((N, D), jnp.float32),
        jax.ShapeDtypeStruct((N, 1), jnp.float32),
    ],
)


def _mm_body(p_ref, u_ref, dinv_ref, b_ref, w_ref, out_ref):
    p = p_ref[...]
    dinv = dinv_ref[...]
    a = jnp.maximum(dinv * (p[0] + p[1] + u_ref[...]) + b_ref[...], 0.0)
    out_ref[...] = dinv * jnp.dot(a, w_ref[...],
                                  preferred_element_type=jnp.float32)


_mm = pl.pallas_call(
    _mm_body,
    grid=(NBLK,),
    in_specs=[
        pl.BlockSpec((NC, R, D), lambda i: (0, i, 0)),
        pl.BlockSpec((R, D), lambda i: (i, 0)),
        pl.BlockSpec((R, 1), lambda i: (i, 0)),
        pl.BlockSpec((1, D), lambda i: (0, 0)),
        pl.BlockSpec((D, D), lambda i: (0, 0)),
    ],
    out_specs=pl.BlockSpec((R, D), lambda i: (i, 0)),
    out_shape=jax.ShapeDtypeStruct((N, D), jnp.float32),
)


def _pool_body(p_ref, u_ref, dinv_ref, b_ref, batch_ref, lw_ref, lb_ref,
               out_ref, pooled):
    i = pl.program_id(0)
    p = p_ref[...]
    dinv = dinv_ref[...]
    a = jnp.maximum(dinv * (p[0] + p[1] + u_ref[...]) + b_ref[...], 0.0)
    bid = batch_ref[...]

    @pl.when(i == 0)
    def _():
        pooled[...] = jnp.full((G, D), -jnp.inf, jnp.float32)

    jmin = jnp.min(bid)
    jmax = jnp.max(bid)
    for j in range(G):
        @pl.when((jmin <= j) & (j <= jmax))
        def _(j=j):
            mj = jnp.max(jnp.where(bid == j, a, -jnp.inf), axis=0,
                         keepdims=True)
            pooled[pl.ds(j, 1), :] = jnp.maximum(pooled[pl.ds(j, 1), :], mj)

    @pl.when(i == NBLK - 1)
    def _():
        out_ref[...] = (jnp.dot(pooled[...], lw_ref[...],
                                preferred_element_type=jnp.float32)
                        + lb_ref[...])


_pool = pl.pallas_call(
    _pool_body,
    grid=(NBLK,),
    in_specs=[
        pl.BlockSpec((NC, R, D), lambda i: (0, i, 0)),
        pl.BlockSpec((R, D), lambda i: (i, 0)),
        pl.BlockSpec((R, 1), lambda i: (i, 0)),
        pl.BlockSpec((1, D), lambda i: (0, 0)),
        pl.BlockSpec((R, 1), lambda i: (i, 0)),
        pl.BlockSpec((D, P), lambda i: (0, 0)),
        pl.BlockSpec((1, P), lambda i: (0, 0)),
    ],
    out_specs=pl.BlockSpec((G, P), lambda i: (0, 0)),
    out_shape=jax.ShapeDtypeStruct((G, P), jnp.float32),
    scratch_shapes=[pltpu.VMEM((G, D), jnp.float32)],
)


def kernel(x, edge_index, batch, W0, b0, W1, b1, W2, b2, lin_W, lin_b):
    src = edge_index[0].reshape(NW, NCH, CHUNK)
    dst = edge_index[1].reshape(NW, NCH, CHUNK)
    degp = _deg(dst)
    u0, dinv = _mm0(x, degp, W0)
    g0 = _agg(u0, src, dst)
    u1 = _mm(g0, u0, dinv, b0.reshape(1, D), W1)
    g1 = _agg(u1, src, dst)
    u2 = _mm(g1, u1, dinv, b1.reshape(1, D), W2)
    g2 = _agg(u2, src, dst)
    out = _pool(g2, u2, dinv, b2.reshape(1, D), batch.reshape(N, 1),
                lin_W, lin_b.reshape(1, P))
    return out


# speedup vs baseline: 30.8099x; 1.0007x over previous
import jax
import jax.numpy as jnp
from jax import lax
from jax.experimental import pallas as pl
from jax.experimental.pallas import tpu as pltpu
from jax.experimental.pallas import tpu_sc as plsc

N = 10000
E = 320000
D = 128
G = 64
P = 16

NC = 2
NS = 16
NW = NC * NS
EPW = E // NW
CHUNK = 80
NCH = EPW // CHUNK
SPLIT = 624
LAST = N - SPLIT * (NS - 1)

R = 2000
NBLK = N // R


def _zero_acc(sid, zbuf, acc):
    @pl.loop(0, pl.cdiv(N // CHUNK, NS))
    def _(k):
        c = k * NS + sid

        @pl.when(c < N // CHUNK)
        def _():
            pltpu.sync_copy(zbuf, acc.at[pl.ds(c * CHUNK, CHUNK)])


def _drain_acc(cid, sid, acc, parts_hbm):
    @pl.when(sid < NS - 1)
    def _():
        pltpu.sync_copy(acc.at[pl.ds(sid * SPLIT, SPLIT)],
                        parts_hbm.at[cid, pl.ds(sid * SPLIT, SPLIT)])

    @pl.when(sid == NS - 1)
    def _():
        pltpu.sync_copy(acc.at[pl.ds((NS - 1) * SPLIT, LAST)],
                        parts_hbm.at[cid, pl.ds((NS - 1) * SPLIT, LAST)])


DW = 16


def _deg_body(dst_hbm, parts_hbm, acc, didx_all, ones_v):
    cid = lax.axis_index("c")
    sid = lax.axis_index("s")
    w = sid * NC + cid

    @pl.loop(0, CHUNK)
    def _zero(i):
        ones_v[i] = jnp.zeros((DW,), jnp.float32)

    _zero_acc(sid, ones_v, acc)
    pltpu.sync_copy(dst_hbm.at[w], didx_all)

    @pl.loop(0, CHUNK)
    def _fill(i):
        ones_v[i] = jnp.full((DW,), 1.0, jnp.float32)

    plsc.subcore_barrier()

    @pl.loop(0, NCH)
    def _step(it):
        pltpu.sync_copy(ones_v, acc.at[didx_all.at[it]], add=True)

    plsc.subcore_barrier()
    _drain_acc(cid, sid, acc, parts_hbm)


_deg = pl.kernel(
    _deg_body,
    out_type=jax.ShapeDtypeStruct((NC, N, DW), jnp.float32),
    mesh=plsc.VectorSubcoreMesh(core_axis_name="c", subcore_axis_name="s"),
    compiler_params=pltpu.CompilerParams(use_tc_tiling_on_sc=False),
    scratch_types=[
        pltpu.VMEM_SHARED((N, DW), jnp.float32),
        pltpu.VMEM((NCH, CHUNK), jnp.int32),
        pltpu.VMEM((CHUNK, DW), jnp.float32),
    ],
)


NBUF = 3


def _agg_body(u_hbm, src_hbm, dst_hbm, parts_hbm, acc, sidx_all, didx_all,
              rows0, rows1, rows2, sem0, sem1, sem2):
    cid = lax.axis_index("c")
    sid = lax.axis_index("s")
    w = sid * NC + cid

    cps = pltpu.async_copy(src_hbm.at[w], sidx_all, sem0)
    cpd = pltpu.async_copy(dst_hbm.at[w], didx_all, sem1)

    @pl.loop(0, CHUNK)
    def _zero(i):
        for j in range(D // 16):
            rows0[i, pl.ds(j * 16, 16)] = jnp.zeros((16,), jnp.float32)

    _zero_acc(sid, rows0, acc)
    cps.wait()
    cpd.wait()
    plsc.subcore_barrier()

    rows = (rows0, rows1, rows2)
    sems = (sem0, sem1, sem2)

    for c in range(NBUF - 1):
        pltpu.async_copy(u_hbm.at[sidx_all.at[c]], rows[c], sems[c])

    @pl.loop(0, (NCH + NBUF - 1) // NBUF)
    def _step(k):
        c0 = NBUF * k
        for b in range(NBUF):
            c = c0 + b
            nxt = c + NBUF - 1
            bn = (b + NBUF - 1) % NBUF

            @pl.when(nxt < NCH)
            def _():
                pltpu.async_copy(u_hbm.at[sidx_all.at[nxt]],
                                 rows[bn], sems[bn])

            @pl.when(c < NCH)
            def _():
                pltpu.make_async_copy(u_hbm.at[sidx_all.at[c]],
                                      rows[b], sems[b]).wait()
                pltpu.sync_copy(rows[b], acc.at[didx_all.at[c]],
                                add=True)

    plsc.subcore_barrier()
    _drain_acc(cid, sid, acc, parts_hbm)


_agg = pl.kernel(
    _agg_body,
    out_type=jax.ShapeDtypeStruct((NC, N, D), jnp.float32),
    mesh=plsc.VectorSubcoreMesh(core_axis_name="c", subcore_axis_name="s"),
    compiler_params=pltpu.CompilerParams(use_tc_tiling_on_sc=False),
    scratch_types=[
        pltpu.VMEM_SHARED((N, D), jnp.float32),
        pltpu.VMEM((NCH, CHUNK), jnp.int32),
        pltpu.VMEM((NCH, CHUNK), jnp.int32),
        pltpu.VMEM((CHUNK, D), jnp.float32),
        pltpu.VMEM((CHUNK, D), jnp.float32),
        pltpu.VMEM((CHUNK, D), jnp.float32),
        pltpu.SemaphoreType.DMA,
        pltpu.SemaphoreType.DMA,
        pltpu.SemaphoreType.DMA,
    ],
)


def _mm0_body(x_ref, degp_ref, w_ref, u_ref, dinv_ref):
    dp = degp_ref[...]
    deg = dp[0, :, 0:1] + dp[1, :, 0:1] + 1.0
    dinv = lax.rsqrt(deg)
    h = jnp.dot(x_ref[...], w_ref[...], preferred_element_type=jnp.float32)
    u_ref[...] = dinv * h
    dinv_ref[...] = dinv


_mm0 = pl.pallas_call(
    _mm0_body,
    grid=(NBLK,),
    in_specs=[
        pl.BlockSpec((R, D), lambda i: (i, 0)),
        pl.BlockSpec((NC, R, DW), lambda i: (0, i, 0)),
        pl.BlockSpec((D, D), lambda i: (0, 0)),
    ],
    out_specs=[
        pl.BlockSpec((R, D), lambda i: (i, 0)),
        pl.BlockSpec((R, 1), lambda i: (i, 0)),
    ],
    out_shape=[
        jax.ShapeDtypeStruct((N, D), jnp.float32),
        jax.ShapeDtypeStruct((N, 1), jnp.float32),
    ],
)


def _mm_body(p_ref, u_ref, dinv_ref, b_ref, w_ref, out_ref):
    p = p_ref[...]
    dinv = dinv_ref[...]
    a = jnp.maximum(dinv * (p[0] + p[1] + u_ref[...]) + b_ref[...], 0.0)
    out_ref[...] = dinv * jnp.dot(a, w_ref[...],
                                  preferred_element_type=jnp.float32)


_mm = pl.pallas_call(
    _mm_body,
    grid=(NBLK,),
    in_specs=[
        pl.BlockSpec((NC, R, D), lambda i: (0, i, 0)),
        pl.BlockSpec((R, D), lambda i: (i, 0)),
        pl.BlockSpec((R, 1), lambda i: (i, 0)),
        pl.BlockSpec((1, D), lambda i: (0, 0)),
        pl.BlockSpec((D, D), lambda i: (0, 0)),
    ],
    out_specs=pl.BlockSpec((R, D), lambda i: (i, 0)),
    out_shape=jax.ShapeDtypeStruct((N, D), jnp.float32),
)


def _pool_body(p_ref, u_ref, dinv_ref, b_ref, batch_ref, lw_ref, lb_ref,
               out_ref, pooled):
    i = pl.program_id(0)
    p = p_ref[...]
    dinv = dinv_ref[...]
    a = jnp.maximum(dinv * (p[0] + p[1] + u_ref[...]) + b_ref[...], 0.0)
    bid = batch_ref[...]

    @pl.when(i == 0)
    def _():
        pooled[...] = jnp.full((G, D), -jnp.inf, jnp.float32)

    jmin = jnp.min(bid)
    jmax = jnp.max(bid)
    for j in range(G):
        @pl.when((jmin <= j) & (j <= jmax))
        def _(j=j):
            mj = jnp.max(jnp.where(bid == j, a, -jnp.inf), axis=0,
                         keepdims=True)
            pooled[pl.ds(j, 1), :] = jnp.maximum(pooled[pl.ds(j, 1), :], mj)

    @pl.when(i == NBLK - 1)
    def _():
        out_ref[...] = (jnp.dot(pooled[...], lw_ref[...],
                                preferred_element_type=jnp.float32)
                        + lb_ref[...])


_pool = pl.pallas_call(
    _pool_body,
    grid=(NBLK,),
    in_specs=[
        pl.BlockSpec((NC, R, D), lambda i: (0, i, 0)),
        pl.BlockSpec((R, D), lambda i: (i, 0)),
        pl.BlockSpec((R, 1), lambda i: (i, 0)),
        pl.BlockSpec((1, D), lambda i: (0, 0)),
        pl.BlockSpec((R, 1), lambda i: (i, 0)),
        pl.BlockSpec((D, P), lambda i: (0, 0)),
        pl.BlockSpec((1, P), lambda i: (0, 0)),
    ],
    out_specs=pl.BlockSpec((G, P), lambda i: (0, 0)),
    out_shape=jax.ShapeDtypeStruct((G, P), jnp.float32),
    scratch_shapes=[pltpu.VMEM((G, D), jnp.float32)],
)


def kernel(x, edge_index, batch, W0, b0, W1, b1, W2, b2, lin_W, lin_b):
    src = edge_index[0].reshape(NW, NCH, CHUNK)
    dst = edge_index[1].reshape(NW, NCH, CHUNK)
    degp = _deg(dst)
    u0, dinv = _mm0(x, degp, W0)
    g0 = _agg(u0, src, dst)
    u1 = _mm(g0, u0, dinv, b0.reshape(1, D), W1)
    g1 = _agg(u1, src, dst)
    u2 = _mm(g1, u1, dinv, b1.reshape(1, D), W2)
    g2 = _agg(u2, src, dst)
    out = _pool(g2, u2, dinv, b2.reshape(1, D), batch.reshape(N, 1),
                lin_W, lin_b.reshape(1, P))
    return out


# speedup vs baseline: 31.3493x; 1.0175x over previous
import jax
import jax.numpy as jnp
from jax import lax
from jax.experimental import pallas as pl
from jax.experimental.pallas import tpu as pltpu
from jax.experimental.pallas import tpu_sc as plsc

N = 10000
E = 320000
D = 128
G = 64
P = 16

NC = 2
NS = 16
NW = NC * NS
EPW = E // NW
CHUNK = 80
NCH = EPW // CHUNK
SPLIT = 624
LAST = N - SPLIT * (NS - 1)

R = 2000
NBLK = N // R


def _zero_acc(sid, zbuf, acc):
    @pl.loop(0, pl.cdiv(N // CHUNK, NS))
    def _(k):
        c = k * NS + sid

        @pl.when(c < N // CHUNK)
        def _():
            pltpu.sync_copy(zbuf, acc.at[pl.ds(c * CHUNK, CHUNK)])


def _drain_acc(cid, sid, acc, parts_hbm):
    @pl.when(sid < NS - 1)
    def _():
        pltpu.sync_copy(acc.at[pl.ds(sid * SPLIT, SPLIT)],
                        parts_hbm.at[cid, pl.ds(sid * SPLIT, SPLIT)])

    @pl.when(sid == NS - 1)
    def _():
        pltpu.sync_copy(acc.at[pl.ds((NS - 1) * SPLIT, LAST)],
                        parts_hbm.at[cid, pl.ds((NS - 1) * SPLIT, LAST)])


DW = 16


def _deg_body(dst_hbm, parts_hbm, acc, didx_all, ones_v, ssem):
    cid = lax.axis_index("c")
    sid = lax.axis_index("s")
    w = sid * NC + cid

    @pl.loop(0, CHUNK)
    def _zero(i):
        ones_v[i] = jnp.zeros((DW,), jnp.float32)

    _zero_acc(sid, ones_v, acc)
    pltpu.sync_copy(dst_hbm.at[w], didx_all)

    @pl.loop(0, CHUNK)
    def _fill(i):
        ones_v[i] = jnp.full((DW,), 1.0, jnp.float32)

    plsc.subcore_barrier()

    K = 5
    @pl.loop(0, NCH // K)
    def _step(kk):
        c0 = K * kk
        for b in range(K):
            pltpu.async_copy(ones_v, acc.at[didx_all.at[c0 + b]], ssem,
                             add=True)
        for b in range(K):
            pltpu.make_async_copy(ones_v, acc.at[didx_all.at[c0 + b]],
                                  ssem).wait()

    plsc.subcore_barrier()
    _drain_acc(cid, sid, acc, parts_hbm)


_deg = pl.kernel(
    _deg_body,
    out_type=jax.ShapeDtypeStruct((NC, N, DW), jnp.float32),
    mesh=plsc.VectorSubcoreMesh(core_axis_name="c", subcore_axis_name="s"),
    compiler_params=pltpu.CompilerParams(use_tc_tiling_on_sc=False),
    scratch_types=[
        pltpu.VMEM_SHARED((N, DW), jnp.float32),
        pltpu.VMEM((NCH, CHUNK), jnp.int32),
        pltpu.VMEM((CHUNK, DW), jnp.float32),
        pltpu.SemaphoreType.DMA,
    ],
)


NBUF = 3


def _agg_body(u_hbm, src_hbm, dst_hbm, parts_hbm, acc, sidx_all, didx_all,
              rows0, rows1, rows2, sem0, sem1, sem2):
    cid = lax.axis_index("c")
    sid = lax.axis_index("s")
    w = sid * NC + cid

    cps = pltpu.async_copy(src_hbm.at[w], sidx_all, sem0)
    cpd = pltpu.async_copy(dst_hbm.at[w], didx_all, sem1)

    @pl.loop(0, CHUNK)
    def _zero(i):
        for j in range(D // 16):
            rows0[i, pl.ds(j * 16, 16)] = jnp.zeros((16,), jnp.float32)

    _zero_acc(sid, rows0, acc)
    cps.wait()
    cpd.wait()
    plsc.subcore_barrier()

    rows = (rows0, rows1, rows2)
    sems = (sem0, sem1, sem2)

    for c in range(NBUF - 1):
        pltpu.async_copy(u_hbm.at[sidx_all.at[c]], rows[c], sems[c])

    @pl.loop(0, (NCH + NBUF - 1) // NBUF)
    def _step(k):
        c0 = NBUF * k
        for b in range(NBUF):
            c = c0 + b
            nxt = c + NBUF - 1
            bn = (b + NBUF - 1) % NBUF

            @pl.when(nxt < NCH)
            def _():
                pltpu.async_copy(u_hbm.at[sidx_all.at[nxt]],
                                 rows[bn], sems[bn])

            @pl.when(c < NCH)
            def _():
                pltpu.make_async_copy(u_hbm.at[sidx_all.at[c]],
                                      rows[b], sems[b]).wait()
                pltpu.sync_copy(rows[b], acc.at[didx_all.at[c]],
                                add=True)

    plsc.subcore_barrier()
    _drain_acc(cid, sid, acc, parts_hbm)


_agg = pl.kernel(
    _agg_body,
    out_type=jax.ShapeDtypeStruct((NC, N, D), jnp.float32),
    mesh=plsc.VectorSubcoreMesh(core_axis_name="c", subcore_axis_name="s"),
    compiler_params=pltpu.CompilerParams(use_tc_tiling_on_sc=False),
    scratch_types=[
        pltpu.VMEM_SHARED((N, D), jnp.float32),
        pltpu.VMEM((NCH, CHUNK), jnp.int32),
        pltpu.VMEM((NCH, CHUNK), jnp.int32),
        pltpu.VMEM((CHUNK, D), jnp.float32),
        pltpu.VMEM((CHUNK, D), jnp.float32),
        pltpu.VMEM((CHUNK, D), jnp.float32),
        pltpu.SemaphoreType.DMA,
        pltpu.SemaphoreType.DMA,
        pltpu.SemaphoreType.DMA,
    ],
)


def _mm0_body(x_ref, degp_ref, w_ref, u_ref, dinv_ref):
    dp = degp_ref[...]
    deg = dp[0, :, 0:1] + dp[1, :, 0:1] + 1.0
    dinv = lax.rsqrt(deg)
    h = jnp.dot(x_ref[...], w_ref[...], preferred_element_type=jnp.float32)
    u_ref[...] = dinv * h
    dinv_ref[...] = dinv


_mm0 = pl.pallas_call(
    _mm0_body,
    grid=(NBLK,),
    in_specs=[
        pl.BlockSpec((R, D), lambda i: (i, 0)),
        pl.BlockSpec((NC, R, DW), lambda i: (0, i, 0)),
        pl.BlockSpec((D, D), lambda i: (0, 0)),
    ],
    out_specs=[
        pl.BlockSpec((R, D), lambda i: (i, 0)),
        pl.BlockSpec((R, 1), lambda i: (i, 0)),
    ],
    out_shape=[
        jax.ShapeDtypeStruct((N, D), jnp.float32),
        jax.ShapeDtypeStruct((N, 1), jnp.float32),
    ],
)


def _mm_body(p_ref, u_ref, dinv_ref, b_ref, w_ref, out_ref):
    p = p_ref[...]
    dinv = dinv_ref[...]
    a = jnp.maximum(dinv * (p[0] + p[1] + u_ref[...]) + b_ref[...], 0.0)
    out_ref[...] = dinv * jnp.dot(a, w_ref[...],
                                  preferred_element_type=jnp.float32)


_mm = pl.pallas_call(
    _mm_body,
    grid=(NBLK,),
    in_specs=[
        pl.BlockSpec((NC, R, D), lambda i: (0, i, 0)),
        pl.BlockSpec((R, D), lambda i: (i, 0)),
        pl.BlockSpec((R, 1), lambda i: (i, 0)),
        pl.BlockSpec((1, D), lambda i: (0, 0)),
        pl.BlockSpec((D, D), lambda i: (0, 0)),
    ],
    out_specs=pl.BlockSpec((R, D), lambda i: (i, 0)),
    out_shape=jax.ShapeDtypeStruct((N, D), jnp.float32),
)


def _pool_body(p_ref, u_ref, dinv_ref, b_ref, batch_ref, lw_ref, lb_ref,
               out_ref, pooled):
    i = pl.program_id(0)
    p = p_ref[...]
    dinv = dinv_ref[...]
    a = jnp.maximum(dinv * (p[0] + p[1] + u_ref[...]) + b_ref[...], 0.0)
    bid = batch_ref[...]

    @pl.when(i == 0)
    def _():
        pooled[...] = jnp.full((G, D), -jnp.inf, jnp.float32)

    jmin = jnp.min(bid)
    jmax = jnp.max(bid)
    for j in range(G):
        @pl.when((jmin <= j) & (j <= jmax))
        def _(j=j):
            mj = jnp.max(jnp.where(bid == j, a, -jnp.inf), axis=0,
                         keepdims=True)
            pooled[pl.ds(j, 1), :] = jnp.maximum(pooled[pl.ds(j, 1), :], mj)

    @pl.when(i == NBLK - 1)
    def _():
        out_ref[...] = (jnp.dot(pooled[...], lw_ref[...],
                                preferred_element_type=jnp.float32)
                        + lb_ref[...])


_pool = pl.pallas_call(
    _pool_body,
    grid=(NBLK,),
    in_specs=[
        pl.BlockSpec((NC, R, D), lambda i: (0, i, 0)),
        pl.BlockSpec((R, D), lambda i: (i, 0)),
        pl.BlockSpec((R, 1), lambda i: (i, 0)),
        pl.BlockSpec((1, D), lambda i: (0, 0)),
        pl.BlockSpec((R, 1), lambda i: (i, 0)),
        pl.BlockSpec((D, P), lambda i: (0, 0)),
        pl.BlockSpec((1, P), lambda i: (0, 0)),
    ],
    out_specs=pl.BlockSpec((G, P), lambda i: (0, 0)),
    out_shape=jax.ShapeDtypeStruct((G, P), jnp.float32),
    scratch_shapes=[pltpu.VMEM((G, D), jnp.float32)],
)


def kernel(x, edge_index, batch, W0, b0, W1, b1, W2, b2, lin_W, lin_b):
    src = edge_index[0].reshape(NW, NCH, CHUNK)
    dst = edge_index[1].reshape(NW, NCH, CHUNK)
    degp = _deg(dst)
    u0, dinv = _mm0(x, degp, W0)
    g0 = _agg(u0, src, dst)
    u1 = _mm(g0, u0, dinv, b0.reshape(1, D), W1)
    g1 = _agg(u1, src, dst)
    u2 = _mm(g1, u1, dinv, b1.reshape(1, D), W2)
    g2 = _agg(u2, src, dst)
    out = _pool(g2, u2, dinv, b2.reshape(1, D), batch.reshape(N, 1),
                lin_W, lin_b.reshape(1, P))
    return out


# speedup vs baseline: 31.4854x; 1.0043x over previous
import jax
import jax.numpy as jnp
from jax import lax
from jax.experimental import pallas as pl
from jax.experimental.pallas import tpu as pltpu
from jax.experimental.pallas import tpu_sc as plsc

N = 10000
E = 320000
D = 128
G = 64
P = 16

NC = 2
NS = 16
NW = NC * NS
EPW = E // NW
CHUNK = 80
NCH = EPW // CHUNK
SPLIT = 624
LAST = N - SPLIT * (NS - 1)

R = 2000
NBLK = N // R


def _zero_acc(sid, zbuf, acc):
    @pl.loop(0, pl.cdiv(N // CHUNK, NS))
    def _(k):
        c = k * NS + sid

        @pl.when(c < N // CHUNK)
        def _():
            pltpu.sync_copy(zbuf, acc.at[pl.ds(c * CHUNK, CHUNK)])


def _drain_acc(cid, sid, acc, parts_hbm):
    @pl.when(sid < NS - 1)
    def _():
        pltpu.sync_copy(acc.at[pl.ds(sid * SPLIT, SPLIT)],
                        parts_hbm.at[cid, pl.ds(sid * SPLIT, SPLIT)])

    @pl.when(sid == NS - 1)
    def _():
        pltpu.sync_copy(acc.at[pl.ds((NS - 1) * SPLIT, LAST)],
                        parts_hbm.at[cid, pl.ds((NS - 1) * SPLIT, LAST)])


DW = 16


def _deg_body(dst_hbm, parts_hbm, acc, didx_all, ones_v, ssem):
    cid = lax.axis_index("c")
    sid = lax.axis_index("s")
    w = sid * NC + cid

    @pl.loop(0, CHUNK)
    def _zero(i):
        ones_v[i] = jnp.zeros((DW,), jnp.float32)

    _zero_acc(sid, ones_v, acc)
    pltpu.sync_copy(dst_hbm.at[w], didx_all)

    @pl.loop(0, CHUNK)
    def _fill(i):
        ones_v[i] = jnp.full((DW,), 1.0, jnp.float32)

    plsc.subcore_barrier()

    K = 5
    @pl.loop(0, NCH // K)
    def _step(kk):
        c0 = K * kk
        for b in range(K):
            pltpu.async_copy(ones_v, acc.at[didx_all.at[c0 + b]], ssem,
                             add=True)
        for b in range(K):
            pltpu.make_async_copy(ones_v, acc.at[didx_all.at[c0 + b]],
                                  ssem).wait()

    plsc.subcore_barrier()
    _drain_acc(cid, sid, acc, parts_hbm)


_deg = pl.kernel(
    _deg_body,
    out_type=jax.ShapeDtypeStruct((NC, N, DW), jnp.float32),
    mesh=plsc.VectorSubcoreMesh(core_axis_name="c", subcore_axis_name="s"),
    compiler_params=pltpu.CompilerParams(use_tc_tiling_on_sc=False),
    scratch_types=[
        pltpu.VMEM_SHARED((N, DW), jnp.float32),
        pltpu.VMEM((NCH, CHUNK), jnp.int32),
        pltpu.VMEM((CHUNK, DW), jnp.float32),
        pltpu.SemaphoreType.DMA,
    ],
)


NBUF = 3


def _agg_body(u_hbm, src_hbm, dst_hbm, parts_hbm, acc, sidx_all, didx_all,
              rows0, rows1, rows2, sem0, sem1, sem2, isem):
    cid = lax.axis_index("c")
    sid = lax.axis_index("s")
    w = sid * NC + cid

    cps = pltpu.async_copy(src_hbm.at[w], sidx_all, isem)
    cpd = pltpu.async_copy(dst_hbm.at[w], didx_all, isem)

    @pl.loop(0, CHUNK)
    def _zero(i):
        for j in range(D // 16):
            rows0[i, pl.ds(j * 16, 16)] = jnp.zeros((16,), jnp.float32)

    cps.wait()
    cpd.wait()

    rows = (rows0, rows1, rows2)
    sems = (sem0, sem1, sem2)

    for c in range(NBUF - 1):
        pltpu.async_copy(u_hbm.at[sidx_all.at[c]],
                         rows[(c + 1) % NBUF], sems[(c + 1) % NBUF])

    _zero_acc(sid, rows0, acc)
    plsc.subcore_barrier()

    @pl.loop(0, (NCH + NBUF - 1) // NBUF)
    def _step(k):
        c0 = NBUF * k
        for b in range(NBUF):
            c = c0 + b
            nxt = c + NBUF - 1
            bc = (b + 1) % NBUF
            bn = (b + NBUF) % NBUF

            @pl.when(nxt < NCH)
            def _():
                pltpu.async_copy(u_hbm.at[sidx_all.at[nxt]],
                                 rows[bn], sems[bn])

            @pl.when(c < NCH)
            def _():
                pltpu.make_async_copy(u_hbm.at[sidx_all.at[c]],
                                      rows[bc], sems[bc]).wait()
                pltpu.sync_copy(rows[bc], acc.at[didx_all.at[c]],
                                add=True)

    plsc.subcore_barrier()
    _drain_acc(cid, sid, acc, parts_hbm)


_agg = pl.kernel(
    _agg_body,
    out_type=jax.ShapeDtypeStruct((NC, N, D), jnp.float32),
    mesh=plsc.VectorSubcoreMesh(core_axis_name="c", subcore_axis_name="s"),
    compiler_params=pltpu.CompilerParams(use_tc_tiling_on_sc=False),
    scratch_types=[
        pltpu.VMEM_SHARED((N, D), jnp.float32),
        pltpu.VMEM((NCH, CHUNK), jnp.int32),
        pltpu.VMEM((NCH, CHUNK), jnp.int32),
        pltpu.VMEM((CHUNK, D), jnp.float32),
        pltpu.VMEM((CHUNK, D), jnp.float32),
        pltpu.VMEM((CHUNK, D), jnp.float32),
        pltpu.SemaphoreType.DMA,
        pltpu.SemaphoreType.DMA,
        pltpu.SemaphoreType.DMA,
        pltpu.SemaphoreType.DMA,
    ],
)


def _mm0_body(x_ref, degp_ref, w_ref, u_ref, dinv_ref):
    dp = degp_ref[...]
    deg = dp[0, :, 0:1] + dp[1, :, 0:1] + 1.0
    dinv = lax.rsqrt(deg)
    h = jnp.dot(x_ref[...], w_ref[...], preferred_element_type=jnp.float32)
    u_ref[...] = dinv * h
    dinv_ref[...] = dinv


_mm0 = pl.pallas_call(
    _mm0_body,
    grid=(NBLK,),
    in_specs=[
        pl.BlockSpec((R, D), lambda i: (i, 0)),
        pl.BlockSpec((NC, R, DW), lambda i: (0, i, 0)),
        pl.BlockSpec((D, D), lambda i: (0, 0)),
    ],
    out_specs=[
        pl.BlockSpec((R, D), lambda i: (i, 0)),
        pl.BlockSpec((R, 1), lambda i: (i, 0)),
    ],
    out_shape=[
        jax.ShapeDtypeStruct((N, D), jnp.float32),
        jax.ShapeDtypeStruct((N, 1), jnp.float32),
    ],
)


def _mm_body(p_ref, u_ref, dinv_ref, b_ref, w_ref, out_ref):
    p = p_ref[...]
    dinv = dinv_ref[...]
    a = jnp.maximum(dinv * (p[0] + p[1] + u_ref[...]) + b_ref[...], 0.0)
    out_ref[...] = dinv * jnp.dot(a, w_ref[...],
                                  preferred_element_type=jnp.float32)


_mm = pl.pallas_call(
    _mm_body,
    grid=(NBLK,),
    in_specs=[
        pl.BlockSpec((NC, R, D), lambda i: (0, i, 0)),
        pl.BlockSpec((R, D), lambda i: (i, 0)),
        pl.BlockSpec((R, 1), lambda i: (i, 0)),
        pl.BlockSpec((1, D), lambda i: (0, 0)),
        pl.BlockSpec((D, D), lambda i: (0, 0)),
    ],
    out_specs=pl.BlockSpec((R, D), lambda i: (i, 0)),
    out_shape=jax.ShapeDtypeStruct((N, D), jnp.float32),
)


def _pool_body(p_ref, u_ref, dinv_ref, b_ref, batch_ref, lw_ref, lb_ref,
               out_ref, pooled):
    i = pl.program_id(0)
    p = p_ref[...]
    dinv = dinv_ref[...]
    a = jnp.maximum(dinv * (p[0] + p[1] + u_ref[...]) + b_ref[...], 0.0)
    bid = batch_ref[...]

    @pl.when(i == 0)
    def _():
        pooled[...] = jnp.full((G, D), -jnp.inf, jnp.float32)

    jmin = jnp.min(bid)
    jmax = jnp.max(bid)
    for j in range(G):
        @pl.when((jmin <= j) & (j <= jmax))
        def _(j=j):
            mj = jnp.max(jnp.where(bid == j, a, -jnp.inf), axis=0,
                         keepdims=True)
            pooled[pl.ds(j, 1), :] = jnp.maximum(pooled[pl.ds(j, 1), :], mj)

    @pl.when(i == NBLK - 1)
    def _():
        out_ref[...] = (jnp.dot(pooled[...], lw_ref[...],
                                preferred_element_type=jnp.float32)
                        + lb_ref[...])


_pool = pl.pallas_call(
    _pool_body,
    grid=(NBLK,),
    in_specs=[
        pl.BlockSpec((NC, R, D), lambda i: (0, i, 0)),
        pl.BlockSpec((R, D), lambda i: (i, 0)),
        pl.BlockSpec((R, 1), lambda i: (i, 0)),
        pl.BlockSpec((1, D), lambda i: (0, 0)),
        pl.BlockSpec((R, 1), lambda i: (i, 0)),
        pl.BlockSpec((D, P), lambda i: (0, 0)),
        pl.BlockSpec((1, P), lambda i: (0, 0)),
    ],
    out_specs=pl.BlockSpec((G, P), lambda i: (0, 0)),
    out_shape=jax.ShapeDtypeStruct((G, P), jnp.float32),
    scratch_shapes=[pltpu.VMEM((G, D), jnp.float32)],
)


def kernel(x, edge_index, batch, W0, b0, W1, b1, W2, b2, lin_W, lin_b):
    src = edge_index[0].reshape(NW, NCH, CHUNK)
    dst = edge_index[1].reshape(NW, NCH, CHUNK)
    degp = _deg(dst)
    u0, dinv = _mm0(x, degp, W0)
    g0 = _agg(u0, src, dst)
    u1 = _mm(g0, u0, dinv, b0.reshape(1, D), W1)
    g1 = _agg(u1, src, dst)
    u2 = _mm(g1, u1, dinv, b1.reshape(1, D), W2)
    g2 = _agg(u2, src, dst)
    out = _pool(g2, u2, dinv, b2.reshape(1, D), batch.reshape(N, 1),
                lin_W, lin_b.reshape(1, P))
    return out


# speedup vs baseline: 32.4390x; 1.0303x over previous
import jax
import jax.numpy as jnp
from jax import lax
from jax.experimental import pallas as pl
from jax.experimental.pallas import tpu as pltpu
from jax.experimental.pallas import tpu_sc as plsc

N = 10000
E = 320000
D = 128
G = 64
P = 16

NC = 2
NS = 16
NW = NC * NS
EPW = E // NW
CHUNK = 80
NCH = EPW // CHUNK
SPLIT = 624
LAST = N - SPLIT * (NS - 1)

R = 2000
NBLK = N // R


def _zero_acc(sid, zbuf, acc):
    @pl.loop(0, pl.cdiv(N // CHUNK, NS))
    def _(k):
        c = k * NS + sid

        @pl.when(c < N // CHUNK)
        def _():
            pltpu.sync_copy(zbuf, acc.at[pl.ds(c * CHUNK, CHUNK)])


def _drain_acc(cid, sid, acc, parts_hbm):
    @pl.when(sid < NS - 1)
    def _():
        pltpu.sync_copy(acc.at[pl.ds(sid * SPLIT, SPLIT)],
                        parts_hbm.at[cid, pl.ds(sid * SPLIT, SPLIT)])

    @pl.when(sid == NS - 1)
    def _():
        pltpu.sync_copy(acc.at[pl.ds((NS - 1) * SPLIT, LAST)],
                        parts_hbm.at[cid, pl.ds((NS - 1) * SPLIT, LAST)])


DW = 16


def _deg_body(ei_hbm, parts_hbm, acc, didx_all, ones_v, ssem):
    cid = lax.axis_index("c")
    sid = lax.axis_index("s")
    w = sid * NC + cid

    @pl.loop(0, CHUNK)
    def _zero(i):
        ones_v[i] = jnp.zeros((DW,), jnp.float32)

    _zero_acc(sid, ones_v, acc)
    pltpu.sync_copy(ei_hbm.at[1, w], didx_all)

    @pl.loop(0, CHUNK)
    def _fill(i):
        ones_v[i] = jnp.full((DW,), 1.0, jnp.float32)

    plsc.subcore_barrier()

    K = 5
    @pl.loop(0, NCH // K)
    def _step(kk):
        c0 = K * kk
        for b in range(K):
            pltpu.async_copy(ones_v, acc.at[didx_all.at[c0 + b]], ssem,
                             add=True)
        for b in range(K):
            pltpu.make_async_copy(ones_v, acc.at[didx_all.at[c0 + b]],
                                  ssem).wait()

    plsc.subcore_barrier()
    _drain_acc(cid, sid, acc, parts_hbm)


_deg = pl.kernel(
    _deg_body,
    out_type=jax.ShapeDtypeStruct((NC, N, DW), jnp.float32),
    mesh=plsc.VectorSubcoreMesh(core_axis_name="c", subcore_axis_name="s"),
    compiler_params=pltpu.CompilerParams(use_tc_tiling_on_sc=False),
    scratch_types=[
        pltpu.VMEM_SHARED((N, DW), jnp.float32),
        pltpu.VMEM((NCH, CHUNK), jnp.int32),
        pltpu.VMEM((CHUNK, DW), jnp.float32),
        pltpu.SemaphoreType.DMA,
    ],
)


NBUF = 3


def _agg_body(u_hbm, ei_hbm, parts_hbm, acc, sidx_all, didx_all,
              rows0, rows1, rows2, sem0, sem1, sem2, isem):
    cid = lax.axis_index("c")
    sid = lax.axis_index("s")
    w = sid * NC + cid

    cps = pltpu.async_copy(ei_hbm.at[0, w], sidx_all, isem)
    cpd = pltpu.async_copy(ei_hbm.at[1, w], didx_all, isem)

    @pl.loop(0, CHUNK)
    def _zero(i):
        for j in range(D // 16):
            rows0[i, pl.ds(j * 16, 16)] = jnp.zeros((16,), jnp.float32)

    cps.wait()
    cpd.wait()

    rows = (rows0, rows1, rows2)
    sems = (sem0, sem1, sem2)

    for c in range(NBUF - 1):
        pltpu.async_copy(u_hbm.at[sidx_all.at[c]],
                         rows[(c + 1) % NBUF], sems[(c + 1) % NBUF])

    _zero_acc(sid, rows0, acc)
    plsc.subcore_barrier()

    @pl.loop(0, (NCH + NBUF - 1) // NBUF)
    def _step(k):
        c0 = NBUF * k
        for b in range(NBUF):
            c = c0 + b
            nxt = c + NBUF - 1
            bc = (b + 1) % NBUF
            bn = (b + NBUF) % NBUF

            @pl.when(nxt < NCH)
            def _():
                pltpu.async_copy(u_hbm.at[sidx_all.at[nxt]],
                                 rows[bn], sems[bn])

            @pl.when(c < NCH)
            def _():
                pltpu.make_async_copy(u_hbm.at[sidx_all.at[c]],
                                      rows[bc], sems[bc]).wait()
                pltpu.sync_copy(rows[bc], acc.at[didx_all.at[c]],
                                add=True)

    plsc.subcore_barrier()
    _drain_acc(cid, sid, acc, parts_hbm)


_agg = pl.kernel(
    _agg_body,
    out_type=jax.ShapeDtypeStruct((NC, N, D), jnp.float32),
    mesh=plsc.VectorSubcoreMesh(core_axis_name="c", subcore_axis_name="s"),
    compiler_params=pltpu.CompilerParams(use_tc_tiling_on_sc=False),
    scratch_types=[
        pltpu.VMEM_SHARED((N, D), jnp.float32),
        pltpu.VMEM((NCH, CHUNK), jnp.int32),
        pltpu.VMEM((NCH, CHUNK), jnp.int32),
        pltpu.VMEM((CHUNK, D), jnp.float32),
        pltpu.VMEM((CHUNK, D), jnp.float32),
        pltpu.VMEM((CHUNK, D), jnp.float32),
        pltpu.SemaphoreType.DMA,
        pltpu.SemaphoreType.DMA,
        pltpu.SemaphoreType.DMA,
        pltpu.SemaphoreType.DMA,
    ],
)


def _mm0_body(x_ref, degp_ref, w_ref, u_ref, dinv_ref):
    dp = degp_ref[...]
    deg = dp[0, :, 0:1] + dp[1, :, 0:1] + 1.0
    dinv = lax.rsqrt(deg)
    h = jnp.dot(x_ref[...], w_ref[...], preferred_element_type=jnp.float32)
    u_ref[...] = dinv * h
    dinv_ref[...] = dinv


_mm0 = pl.pallas_call(
    _mm0_body,
    grid=(NBLK,),
    in_specs=[
        pl.BlockSpec((R, D), lambda i: (i, 0)),
        pl.BlockSpec((NC, R, DW), lambda i: (0, i, 0)),
        pl.BlockSpec((D, D), lambda i: (0, 0)),
    ],
    out_specs=[
        pl.BlockSpec((R, D), lambda i: (i, 0)),
        pl.BlockSpec((R, 1), lambda i: (i, 0)),
    ],
    out_shape=[
        jax.ShapeDtypeStruct((N, D), jnp.float32),
        jax.ShapeDtypeStruct((N, 1), jnp.float32),
    ],
)


def _mm_body(p_ref, u_ref, dinv_ref, b_ref, w_ref, out_ref):
    p = p_ref[...]
    dinv = dinv_ref[...]
    a = jnp.maximum(dinv * (p[0] + p[1] + u_ref[...]) + b_ref[...], 0.0)
    out_ref[...] = dinv * jnp.dot(a, w_ref[...],
                                  preferred_element_type=jnp.float32)


_mm = pl.pallas_call(
    _mm_body,
    grid=(NBLK,),
    in_specs=[
        pl.BlockSpec((NC, R, D), lambda i: (0, i, 0)),
        pl.BlockSpec((R, D), lambda i: (i, 0)),
        pl.BlockSpec((R, 1), lambda i: (i, 0)),
        pl.BlockSpec((1, D), lambda i: (0, 0)),
        pl.BlockSpec((D, D), lambda i: (0, 0)),
    ],
    out_specs=pl.BlockSpec((R, D), lambda i: (i, 0)),
    out_shape=jax.ShapeDtypeStruct((N, D), jnp.float32),
)


def _pool_body(p_ref, u_ref, dinv_ref, b_ref, batch_ref, lw_ref, lb_ref,
               out_ref, pooled):
    i = pl.program_id(0)
    p = p_ref[...]
    dinv = dinv_ref[...]
    a = jnp.maximum(dinv * (p[0] + p[1] + u_ref[...]) + b_ref[...], 0.0)
    bid = batch_ref[...]

    @pl.when(i == 0)
    def _():
        pooled[...] = jnp.full((G, D), -jnp.inf, jnp.float32)

    jmin = jnp.min(bid)
    jmax = jnp.max(bid)
    for j in range(G):
        @pl.when((jmin <= j) & (j <= jmax))
        def _(j=j):
            mj = jnp.max(jnp.where(bid == j, a, -jnp.inf), axis=0,
                         keepdims=True)
            pooled[pl.ds(j, 1), :] = jnp.maximum(pooled[pl.ds(j, 1), :], mj)

    @pl.when(i == NBLK - 1)
    def _():
        out_ref[...] = (jnp.dot(pooled[...], lw_ref[...],
                                preferred_element_type=jnp.float32)
                        + lb_ref[...])


_pool = pl.pallas_call(
    _pool_body,
    grid=(NBLK,),
    in_specs=[
        pl.BlockSpec((NC, R, D), lambda i: (0, i, 0)),
        pl.BlockSpec((R, D), lambda i: (i, 0)),
        pl.BlockSpec((R, 1), lambda i: (i, 0)),
        pl.BlockSpec((1, D), lambda i: (0, 0)),
        pl.BlockSpec((R, 1), lambda i: (i, 0)),
        pl.BlockSpec((D, P), lambda i: (0, 0)),
        pl.BlockSpec((1, P), lambda i: (0, 0)),
    ],
    out_specs=pl.BlockSpec((G, P), lambda i: (0, 0)),
    out_shape=jax.ShapeDtypeStruct((G, P), jnp.float32),
    scratch_shapes=[pltpu.VMEM((G, D), jnp.float32)],
)


def kernel(x, edge_index, batch, W0, b0, W1, b1, W2, b2, lin_W, lin_b):
    ei = edge_index.reshape(2, NW, NCH, CHUNK)
    degp = _deg(ei)
    u0, dinv = _mm0(x, degp, W0)
    g0 = _agg(u0, ei)
    u1 = _mm(g0, u0, dinv, b0.reshape(1, D), W1)
    g1 = _agg(u1, ei)
    u2 = _mm(g1, u1, dinv, b1.reshape(1, D), W2)
    g2 = _agg(u2, ei)
    out = _pool(g2, u2, dinv, b2.reshape(1, D), batch.reshape(N, 1),
                lin_W, lin_b.reshape(1, P))
    return out
